# Initial kernel scaffold; baseline (speedup 1.0000x reference)
import functools

import jax
import jax.numpy as jnp
from jax import lax
from jax.experimental import pallas as pl
from jax.experimental.pallas import tpu as pltpu
from jax.experimental.pallas import tpu_sc as plsc

N = 50000
E = 800000
IN_DIM = 64
D = 70
DP = 80
G = 5
NW = 32
BK = 128
NBLK = 196
EPT = NBLK * BK
EPAD = NW * EPT
NT = 50016
RPS = NT // 16


BN_NODE = 2000
NSTEPS = N // BN_NODE


def _pack_tables(tabs):
  Bh = tabs[:, DP:2 * DP]
  Dh = tabs[:, 2 * DP:3 * DP]
  Eh = tabs[:, 3 * DP:4 * DP]
  src = jnp.stack([
      jnp.concatenate([Bh[:, 16 * g:16 * (g + 1)], Dh[:, 16 * g:16 * (g + 1)]],
                      axis=1) for g in range(G)], axis=0)
  dst = jnp.stack([Eh[:, 16 * g:16 * (g + 1)] for g in range(G)], axis=0)
  return src, dst


def _entry_body(x_ref, wh_ref, bh_ref, wp_ref, bp_ref,
                h0_ref, a_ref, s_ref, d_ref):
  h0 = jnp.dot(x_ref[...], wh_ref[...],
               preferred_element_type=jnp.float32) + bh_ref[...]
  tabs = jnp.dot(h0, wp_ref[...],
                 preferred_element_type=jnp.float32) + bp_ref[...]
  h0_ref[...] = h0
  a_ref[...] = tabs[:, :DP]
  s, d = _pack_tables(tabs)
  s_ref[...] = s
  d_ref[...] = d


def _entry_call(x, wh, bh, wp, bp):
  return pl.pallas_call(
      _entry_body,
      grid=(NSTEPS,),
      in_specs=[
          pl.BlockSpec((BN_NODE, IN_DIM), lambda i: (i, 0)),
          pl.BlockSpec((IN_DIM, DP), lambda i: (0, 0)),
          pl.BlockSpec((1, DP), lambda i: (0, 0)),
          pl.BlockSpec((DP, 4 * DP), lambda i: (0, 0)),
          pl.BlockSpec((1, 4 * DP), lambda i: (0, 0)),
      ],
      out_specs=[
          pl.BlockSpec((BN_NODE, DP), lambda i: (i, 0)),
          pl.BlockSpec((BN_NODE, DP), lambda i: (i, 0)),
          pl.BlockSpec((G, BN_NODE, 32), lambda i: (0, i, 0)),
          pl.BlockSpec((G, BN_NODE, 16), lambda i: (0, i, 0)),
      ],
      out_shape=[
          jax.ShapeDtypeStruct((N, DP), jnp.float32),
          jax.ShapeDtypeStruct((N, DP), jnp.float32),
          jax.ShapeDtypeStruct((G, N, 32), jnp.float32),
          jax.ShapeDtypeStruct((G, N, 16), jnp.float32),
      ],
  )(x, wh, bh, wp, bp)


def _hnew_body(a_ref, nd_ref, snn_ref, h_ref, st_ref, acc_ref):
  i = pl.program_id(0)
  num = jnp.concatenate(
      [nd_ref[0, g, :, 0:16] + nd_ref[1, g, :, 0:16] for g in range(G)],
      axis=1)
  den = jnp.concatenate(
      [nd_ref[0, g, :, 16:32] + nd_ref[1, g, :, 16:32] for g in range(G)],
      axis=1)
  hnew = (a_ref[...] + num / (den + 1e-6)) * snn_ref[...]
  h_ref[...] = hnew

  @pl.when(i == 0)
  def _():
    acc_ref[...] = jnp.zeros_like(acc_ref)

  acc_ref[0:1, :] += jnp.sum(hnew, axis=0, keepdims=True)
  acc_ref[1:2, :] += jnp.sum(hnew * hnew, axis=0, keepdims=True)

  @pl.when(i == NSTEPS - 1)
  def _():
    st_ref[...] = acc_ref[...]


def _hnew_call(a, nd, snn):
  return pl.pallas_call(
      _hnew_body,
      grid=(NSTEPS,),
      in_specs=[
          pl.BlockSpec((BN_NODE, DP), lambda i: (i, 0)),
          pl.BlockSpec((2, G, BN_NODE, 32), lambda i: (0, 0, i, 0)),
          pl.BlockSpec((BN_NODE, 1), lambda i: (i, 0)),
      ],
      out_specs=[
          pl.BlockSpec((BN_NODE, DP), lambda i: (i, 0)),
          pl.BlockSpec((2, DP), lambda i: (0, 0)),
      ],
      out_shape=[
          jax.ShapeDtypeStruct((N, DP), jnp.float32),
          jax.ShapeDtypeStruct((2, DP), jnp.float32),
      ],
      scratch_shapes=[pltpu.VMEM((2, DP), jnp.float32)],
  )(a, nd, snn)


def _hfin_body(hprev_ref, hnew_ref, st_ref, g_ref, b_ref, wp_ref, bp_ref,
               h_ref, a_ref, s_ref, d_ref):
  m = st_ref[0:1, :] / N
  v = st_ref[1:2, :] / N - m * m
  hn = g_ref[...] * (hnew_ref[...] - m) * lax.rsqrt(v + 1e-5) + b_ref[...]
  h1 = hprev_ref[...] + jnp.maximum(hn, 0.0)
  h_ref[...] = h1
  tabs = jnp.dot(h1, wp_ref[...],
                 preferred_element_type=jnp.float32) + bp_ref[...]
  a_ref[...] = tabs[:, :DP]
  s, d = _pack_tables(tabs)
  s_ref[...] = s
  d_ref[...] = d


def _hfin_call(hprev, hnew, st, g, b, wp, bp):
  return pl.pallas_call(
      _hfin_body,
      grid=(NSTEPS,),
      in_specs=[
          pl.BlockSpec((BN_NODE, DP), lambda i: (i, 0)),
          pl.BlockSpec((BN_NODE, DP), lambda i: (i, 0)),
          pl.BlockSpec((2, DP), lambda i: (0, 0)),
          pl.BlockSpec((1, DP), lambda i: (0, 0)),
          pl.BlockSpec((1, DP), lambda i: (0, 0)),
          pl.BlockSpec((DP, 4 * DP), lambda i: (0, 0)),
          pl.BlockSpec((1, 4 * DP), lambda i: (0, 0)),
      ],
      out_specs=[
          pl.BlockSpec((BN_NODE, DP), lambda i: (i, 0)),
          pl.BlockSpec((BN_NODE, DP), lambda i: (i, 0)),
          pl.BlockSpec((G, BN_NODE, 32), lambda i: (0, i, 0)),
          pl.BlockSpec((G, BN_NODE, 16), lambda i: (0, i, 0)),
      ],
      out_shape=[
          jax.ShapeDtypeStruct((N, DP), jnp.float32),
          jax.ShapeDtypeStruct((N, DP), jnp.float32),
          jax.ShapeDtypeStruct((G, N, 32), jnp.float32),
          jax.ShapeDtypeStruct((G, N, 16), jnp.float32),
      ],
  )(hprev, hnew, st, g, b, wp, bp)


BE = 2048
ESTEPS = EPAD // BE


def _ce2_body(t_ref, ef_ref, st_ref, werow_ref, be_ref, g1_ref, b1_ref,
              cw_ref, cb_ref, out_ref):
  parts = []
  for g in range(G):
    sums = jnp.sum(st_ref[:, :, g, 0:16], axis=(0, 1))
    sqs = jnp.sum(st_ref[:, :, g, 16:32], axis=(0, 1))
    m = sums / E
    v = sqs / E - m * m
    sl = slice(16 * g, 16 * (g + 1))
    tg = t_ref[:, sl]
    bn = g1_ref[0:1, sl] * (tg - m[None, :]) * lax.rsqrt(v + 1e-5)[None, :] \
        + b1_ref[0:1, sl]
    e0g = ef_ref[...] * werow_ref[0:1, sl] + be_ref[0:1, sl]
    parts.append(e0g + jnp.maximum(bn, 0.0))
  e1 = jnp.concatenate(parts, axis=1)
  out_ref[...] = jnp.dot(e1, cw_ref[...],
                         preferred_element_type=jnp.float32) + cb_ref[...]


def _ce2_call(t, ef, st, werow, be, g1, b1, cw, cb):
  return pl.pallas_call(
      _ce2_body,
      grid=(ESTEPS,),
      in_specs=[
          pl.BlockSpec((BE, DP), lambda i: (i, 0)),
          pl.BlockSpec((BE, 1), lambda i: (i, 0)),
          pl.BlockSpec((2, 16, G, 32), lambda i: (0, 0, 0, 0)),
          pl.BlockSpec((1, DP), lambda i: (0, 0)),
          pl.BlockSpec((1, DP), lambda i: (0, 0)),
          pl.BlockSpec((1, DP), lambda i: (0, 0)),
          pl.BlockSpec((1, DP), lambda i: (0, 0)),
          pl.BlockSpec((DP, DP), lambda i: (0, 0)),
          pl.BlockSpec((1, DP), lambda i: (0, 0)),
      ],
      out_specs=pl.BlockSpec((BE, DP), lambda i: (i, 0)),
      out_shape=jax.ShapeDtypeStruct((EPAD, DP), jnp.float32),
  )(t, ef, st, werow, be, g1, b1, cw, cb)


def _final_body(hprev_ref, hnew_ref, st_ref, g_ref, b_ref, out_ref, acc_ref):
  i = pl.program_id(0)
  m = st_ref[0:1, :] / N
  v = st_ref[1:2, :] / N - m * m
  hn = g_ref[...] * (hnew_ref[...] - m) * lax.rsqrt(v + 1e-5) + b_ref[...]
  h2 = hprev_ref[...] + jnp.maximum(hn, 0.0)

  @pl.when(i == 0)
  def _():
    acc_ref[...] = jnp.zeros_like(acc_ref)

  acc_ref[...] += jnp.sum(h2, axis=0, keepdims=True)

  @pl.when(i == NSTEPS - 1)
  def _():
    out_ref[...] = acc_ref[...] / N


def _final_call(hprev, hnew, st, g, b):
  return pl.pallas_call(
      _final_body,
      grid=(NSTEPS,),
      in_specs=[
          pl.BlockSpec((BN_NODE, DP), lambda i: (i, 0)),
          pl.BlockSpec((BN_NODE, DP), lambda i: (i, 0)),
          pl.BlockSpec((2, DP), lambda i: (0, 0)),
          pl.BlockSpec((1, DP), lambda i: (0, 0)),
          pl.BlockSpec((1, DP), lambda i: (0, 0)),
      ],
      out_specs=pl.BlockSpec((1, DP), lambda i: (0, 0)),
      out_shape=jax.ShapeDtypeStruct((1, DP), jnp.float32),
      scratch_shapes=[pltpu.VMEM((1, DP), jnp.float32)],
  )(hprev, hnew, st, g, b)


_MESH = plsc.VectorSubcoreMesh(core_axis_name="c", subcore_axis_name="s")


def _sc_layer1(sidx, didx, efv, snv, stab, dtab, u, w, zrow):
  out_type = (
      jax.ShapeDtypeStruct((2, G, NT, 32), jnp.float32),
      jax.ShapeDtypeStruct((EPAD, DP), jnp.float32),
      jax.ShapeDtypeStruct((2, 16, G, 32), jnp.float32),
  )
  scratch = [
      pltpu.VMEM_SHARED((NT, 32), jnp.float32),
      pltpu.VMEM((NBLK, BK), jnp.int32),
      pltpu.VMEM((NBLK, BK), jnp.int32),
      pltpu.VMEM((NBLK, BK), jnp.float32),
      pltpu.VMEM((NBLK, BK), jnp.float32),
      pltpu.VMEM((BK,), jnp.int32),
      pltpu.VMEM((BK,), jnp.int32),
      pltpu.VMEM((BK, 32), jnp.float32),
      pltpu.VMEM((BK, 16), jnp.float32),
      pltpu.VMEM((BK, 32), jnp.float32),
      pltpu.VMEM((BK, 16), jnp.float32),
      pltpu.VMEM((DP,), jnp.float32),
      pltpu.VMEM((DP,), jnp.float32),
      pltpu.VMEM((32,), jnp.float32),
  ]

  @functools.partial(pl.kernel, out_type=out_type, mesh=_MESH,
                     scratch_types=scratch)
  def body(sidx_h, didx_h, ef_h, sn_h, stab_h, dtab_h, u_h, w_h, z_h,
           nd_h, t_h, st_h,
           acc, sidx_v, didx_v, ef_v, sn_v, sgi_v, gdi_v, Sv, Dv, Pv, Tv,
           uv, wv, stbuf):
    c = lax.axis_index("c")
    s = lax.axis_index("s")
    wid = c * 16 + s
    ebase = wid * EPT
    pltpu.sync_copy(sidx_h.at[wid], sidx_v)
    pltpu.sync_copy(didx_h.at[wid], didx_v)
    pltpu.sync_copy(ef_h.at[wid], ef_v)
    pltpu.sync_copy(sn_h.at[wid], sn_v)
    pltpu.sync_copy(u_h, uv)
    pltpu.sync_copy(w_h, wv)

    for g in range(G):
      pltpu.sync_copy(z_h, acc.at[pl.ds(s * RPS, RPS)])
      plsc.subcore_barrier()
      ug = uv[pl.ds(16 * g, 16)]
      wg = wv[pl.ds(16 * g, 16)]

      def blk(b, carry):
        for kk in range(BK // 16):
          sgi_v[pl.ds(kk * 16, 16)] = (
              sidx_v[b, pl.ds(kk * 16, 16)] + g * N)
          gdi_v[pl.ds(kk * 16, 16)] = (
              didx_v[b, pl.ds(kk * 16, 16)] + g * NT)
        pltpu.sync_copy(stab_h.at[sgi_v], Sv)
        pltpu.sync_copy(dtab_h.at[gdi_v], Dv)

        def edge(i, ec):
          ssum, ssq = ec
          bh = Sv[i, pl.ds(0, 16)]
          dh = Sv[i, pl.ds(16, 16)]
          eh = Dv[i, pl.ds(0, 16)]
          ce = ef_v[b, i] * ug + wg
          en = ce + dh + eh
          sig = 1.0 / (1.0 + jnp.exp(-en))
          Pv[i, pl.ds(0, 16)] = sig * bh
          Pv[i, pl.ds(16, 16)] = sig
          t = en * sn_v[b, i]
          Tv[i, pl.ds(0, 16)] = t
          return ssum + t, ssq + t * t

        carry = lax.fori_loop(0, BK, edge, carry)
        pltpu.sync_copy(Pv, acc.at[didx_v.at[b]], add=True)
        pltpu.sync_copy(
            Tv, t_h.at[pl.ds(ebase + b * BK, BK), pl.ds(16 * g, 16)])
        return carry

      z16 = jnp.zeros((16,), jnp.float32)
      ssum, ssq = lax.fori_loop(0, NBLK, blk, (z16, z16))
      stbuf[pl.ds(0, 16)] = ssum
      stbuf[pl.ds(16, 16)] = ssq
      pltpu.sync_copy(stbuf, st_h.at[c, s, g])
      plsc.subcore_barrier()
      pltpu.sync_copy(acc.at[pl.ds(s * RPS, RPS)],
                      nd_h.at[c, g, pl.ds(s * RPS, RPS)])

  return body(sidx, didx, efv, snv, stab, dtab, u, w, zrow)


def _sc_layer2(sidx, didx, ce, stab, dtab, zrow):
  out_type = jax.ShapeDtypeStruct((2, G, NT, 32), jnp.float32)
  scratch = [
      pltpu.VMEM_SHARED((NT, 32), jnp.float32),
      pltpu.VMEM((NBLK, BK), jnp.int32),
      pltpu.VMEM((NBLK, BK), jnp.int32),
      pltpu.VMEM((BK,), jnp.int32),
      pltpu.VMEM((BK,), jnp.int32),
      pltpu.VMEM((BK, 32), jnp.float32),
      pltpu.VMEM((BK, 16), jnp.float32),
      pltpu.VMEM((BK, 16), jnp.float32),
      pltpu.VMEM((BK, 32), jnp.float32),
  ]

  @functools.partial(pl.kernel, out_type=out_type, mesh=_MESH,
                     scratch_types=scratch)
  def body(sidx_h, didx_h, ce_h, stab_h, dtab_h, z_h, nd_h,
           acc, sidx_v, didx_v, sgi_v, gdi_v, Sv, Dv, Cv, Pv):
    c = lax.axis_index("c")
    s = lax.axis_index("s")
    wid = c * 16 + s
    ebase = wid * EPT
    pltpu.sync_copy(sidx_h.at[wid], sidx_v)
    pltpu.sync_copy(didx_h.at[wid], didx_v)

    for g in range(G):
      pltpu.sync_copy(z_h, acc.at[pl.ds(s * RPS, RPS)])
      plsc.subcore_barrier()

      def blk(b, carry):
        for kk in range(BK // 16):
          sgi_v[pl.ds(kk * 16, 16)] = (
              sidx_v[b, pl.ds(kk * 16, 16)] + g * N)
          gdi_v[pl.ds(kk * 16, 16)] = (
              didx_v[b, pl.ds(kk * 16, 16)] + g * NT)
        pltpu.sync_copy(stab_h.at[sgi_v], Sv)
        pltpu.sync_copy(dtab_h.at[gdi_v], Dv)
        pltpu.sync_copy(
            ce_h.at[pl.ds(ebase + b * BK, BK), pl.ds(16 * g, 16)], Cv)

        def edge(i, ec):
          bh = Sv[i, pl.ds(0, 16)]
          dh = Sv[i, pl.ds(16, 16)]
          eh = Dv[i, pl.ds(0, 16)]
          en = Cv[i, pl.ds(0, 16)] + dh + eh
          sig = 1.0 / (1.0 + jnp.exp(-en))
          Pv[i, pl.ds(0, 16)] = sig * bh
          Pv[i, pl.ds(16, 16)] = sig
          return ec

        lax.fori_loop(0, BK, edge, 0)
        pltpu.sync_copy(Pv, acc.at[didx_v.at[b]], add=True)
        return carry

      lax.fori_loop(0, NBLK, blk, 0)
      plsc.subcore_barrier()
      pltpu.sync_copy(acc.at[pl.ds(s * RPS, RPS)],
                      nd_h.at[c, g, pl.ds(s * RPS, RPS)])

  return body(sidx, didx, ce, stab, dtab, zrow)


def _padcol(a, width):
  return jnp.pad(a, ((0, 0), (0, width - a.shape[1])))


def _padvec(v, width):
  return jnp.pad(v, (0, width - v.shape[0]))


@jax.jit
def _run(edge_index, nodes_feat, edges_feat, snorm_n, snorm_e, params):
  f32 = jnp.float32
  src = edge_index[0]
  dst = edge_index[1]

  whp = _padcol(params["Wh"], DP)
  bhp = _padvec(params["bh"], DP)[None, :]
  lp = params["layers"]

  def pack_layer(p):
    wp = jnp.concatenate(
        [_padcol(p[k + "_W"], DP) for k in "ABDE"], axis=1)
    wp = jnp.pad(wp, ((0, DP - D), (0, 0)))
    bp = jnp.concatenate([_padvec(p[k + "_b"], DP) for k in "ABDE"])[None, :]
    return wp, bp

  wp1, bp1 = pack_layer(lp[0])
  wp2, bp2 = pack_layer(lp[1])

  we_row = _padvec(params["We"][0], DP)[None, :]
  bep = _padvec(params["be"], DP)[None, :]
  u1 = _padvec(params["We"][0] @ lp[0]["C_W"], DP)
  w1 = _padvec(params["be"] @ lp[0]["C_W"] + lp[0]["C_b"], DP)
  c2w = jnp.pad(_padcol(lp[1]["C_W"], DP), ((0, DP - D), (0, 0)))
  c2b = _padvec(lp[1]["C_b"], DP)[None, :]
  bnh1_g = _padvec(lp[0]["bnh_g"], DP)[None, :]
  bnh1_b = _padvec(lp[0]["bnh_b"], DP)[None, :]
  bne1_g = _padvec(lp[0]["bne_g"], DP)[None, :]
  bne1_b = _padvec(lp[0]["bne_b"], DP)[None, :]
  bnh2_g = _padvec(lp[1]["bnh_g"], DP)[None, :]
  bnh2_b = _padvec(lp[1]["bnh_b"], DP)[None, :]

  npad = EPAD - E
  srcp = jnp.concatenate([src, jnp.zeros((npad,), jnp.int32)]).reshape(
      NW, NBLK, BK)
  dstp = jnp.concatenate([dst, jnp.full((npad,), N, jnp.int32)]).reshape(
      NW, NBLK, BK)
  efp = jnp.concatenate([edges_feat[:, 0],
                         jnp.zeros((npad,), f32)]).reshape(NW, NBLK, BK)
  snp = jnp.concatenate([snorm_e[:, 0],
                         jnp.zeros((npad,), f32)]).reshape(NW, NBLK, BK)
  ef2d = jnp.concatenate([edges_feat[:, 0],
                          jnp.zeros((npad,), f32)])[:, None]
  zrow = jnp.zeros((RPS, 32), f32)

  h0, a1, s1, d1 = _entry_call(nodes_feat, whp, bhp, wp1, bp1)
  stab1 = s1.reshape(G * N, 32)
  dtab1 = jnp.pad(d1, ((0, 0), (0, NT - N), (0, 0))).reshape(G * NT, 16)
  nd1, t1, st1 = _sc_layer1(srcp, dstp, efp, snp, stab1, dtab1, u1, w1, zrow)
  hnew1, hst1 = _hnew_call(a1, nd1, snorm_n)
  h1, a2, s2, d2 = _hfin_call(h0, hnew1, hst1, bnh1_g, bnh1_b, wp2, bp2)

  ce2 = _ce2_call(t1, ef2d, st1, we_row, bep, bne1_g, bne1_b, c2w, c2b)
  stab2 = s2.reshape(G * N, 32)
  dtab2 = jnp.pad(d2, ((0, 0), (0, NT - N), (0, 0))).reshape(G * NT, 16)
  nd2 = _sc_layer2(srcp, dstp, ce2, stab2, dtab2, zrow)
  hnew2, hst2 = _hnew_call(a2, nd2, snorm_n)
  out = _final_call(h1, hnew2, hst2, bnh2_g, bnh2_b)
  return out[:, :D]


def kernel(edge_index, nodes_feat, edges_feat, nodes_num_norm_sqrt,
           edges_num_norm_sqrt, params):
  return _run(edge_index, nodes_feat, edges_feat, nodes_num_norm_sqrt,
              edges_num_norm_sqrt, params)


# speedup vs baseline: 1.1469x; 1.1469x over previous
import functools

import jax
import jax.numpy as jnp
from jax import lax
from jax.experimental import pallas as pl
from jax.experimental.pallas import tpu as pltpu
from jax.experimental.pallas import tpu_sc as plsc

N = 50000
E = 800000
IN_DIM = 64
D = 70
DP = 80
G = 5
NW = 32
BK = 128
NBLK = 196
EPT = NBLK * BK
EPAD = NW * EPT
NT = 50016
RPS = NT // 16


BN_NODE = 2000
NSTEPS = N // BN_NODE


def _pack_tables(tabs):
  Bh = tabs[:, DP:2 * DP]
  Dh = tabs[:, 2 * DP:3 * DP]
  Eh = tabs[:, 3 * DP:4 * DP]
  src = jnp.stack([
      jnp.concatenate([Bh[:, 16 * g:16 * (g + 1)], Dh[:, 16 * g:16 * (g + 1)]],
                      axis=1) for g in range(G)], axis=0)
  dst = jnp.stack([Eh[:, 16 * g:16 * (g + 1)] for g in range(G)], axis=0)
  return src, dst


def _entry_body(x_ref, wh_ref, bh_ref, wp_ref, bp_ref,
                h0_ref, a_ref, s_ref, d_ref):
  h0 = jnp.dot(x_ref[...], wh_ref[...],
               preferred_element_type=jnp.float32) + bh_ref[...]
  tabs = jnp.dot(h0, wp_ref[...],
                 preferred_element_type=jnp.float32) + bp_ref[...]
  h0_ref[...] = h0
  a_ref[...] = tabs[:, :DP]
  s, d = _pack_tables(tabs)
  s_ref[...] = s
  d_ref[...] = d


def _entry_call(x, wh, bh, wp, bp):
  return pl.pallas_call(
      _entry_body,
      grid=(NSTEPS,),
      in_specs=[
          pl.BlockSpec((BN_NODE, IN_DIM), lambda i: (i, 0)),
          pl.BlockSpec((IN_DIM, DP), lambda i: (0, 0)),
          pl.BlockSpec((1, DP), lambda i: (0, 0)),
          pl.BlockSpec((DP, 4 * DP), lambda i: (0, 0)),
          pl.BlockSpec((1, 4 * DP), lambda i: (0, 0)),
      ],
      out_specs=[
          pl.BlockSpec((BN_NODE, DP), lambda i: (i, 0)),
          pl.BlockSpec((BN_NODE, DP), lambda i: (i, 0)),
          pl.BlockSpec((G, BN_NODE, 32), lambda i: (0, i, 0)),
          pl.BlockSpec((G, BN_NODE, 16), lambda i: (0, i, 0)),
      ],
      out_shape=[
          jax.ShapeDtypeStruct((N, DP), jnp.float32),
          jax.ShapeDtypeStruct((N, DP), jnp.float32),
          jax.ShapeDtypeStruct((G, N, 32), jnp.float32),
          jax.ShapeDtypeStruct((G, N, 16), jnp.float32),
      ],
  )(x, wh, bh, wp, bp)


def _hnew_body(a_ref, nd_ref, snn_ref, h_ref, st_ref, acc_ref):
  i = pl.program_id(0)
  num = jnp.concatenate(
      [nd_ref[0, g, :, 0:16] + nd_ref[1, g, :, 0:16] for g in range(G)],
      axis=1)
  den = jnp.concatenate(
      [nd_ref[0, g, :, 16:32] + nd_ref[1, g, :, 16:32] for g in range(G)],
      axis=1)
  hnew = (a_ref[...] + num / (den + 1e-6)) * snn_ref[...]
  h_ref[...] = hnew

  @pl.when(i == 0)
  def _():
    acc_ref[...] = jnp.zeros_like(acc_ref)

  acc_ref[0:1, :] += jnp.sum(hnew, axis=0, keepdims=True)
  acc_ref[1:2, :] += jnp.sum(hnew * hnew, axis=0, keepdims=True)

  @pl.when(i == NSTEPS - 1)
  def _():
    st_ref[...] = acc_ref[...]


def _hnew_call(a, nd, snn):
  return pl.pallas_call(
      _hnew_body,
      grid=(NSTEPS,),
      in_specs=[
          pl.BlockSpec((BN_NODE, DP), lambda i: (i, 0)),
          pl.BlockSpec((2, G, BN_NODE, 32), lambda i: (0, 0, i, 0)),
          pl.BlockSpec((BN_NODE, 1), lambda i: (i, 0)),
      ],
      out_specs=[
          pl.BlockSpec((BN_NODE, DP), lambda i: (i, 0)),
          pl.BlockSpec((2, DP), lambda i: (0, 0)),
      ],
      out_shape=[
          jax.ShapeDtypeStruct((N, DP), jnp.float32),
          jax.ShapeDtypeStruct((2, DP), jnp.float32),
      ],
      scratch_shapes=[pltpu.VMEM((2, DP), jnp.float32)],
  )(a, nd, snn)


def _hfin_body(hprev_ref, hnew_ref, st_ref, g_ref, b_ref, wp_ref, bp_ref,
               h_ref, a_ref, s_ref, d_ref):
  m = st_ref[0:1, :] / N
  v = st_ref[1:2, :] / N - m * m
  hn = g_ref[...] * (hnew_ref[...] - m) * lax.rsqrt(v + 1e-5) + b_ref[...]
  h1 = hprev_ref[...] + jnp.maximum(hn, 0.0)
  h_ref[...] = h1
  tabs = jnp.dot(h1, wp_ref[...],
                 preferred_element_type=jnp.float32) + bp_ref[...]
  a_ref[...] = tabs[:, :DP]
  s, d = _pack_tables(tabs)
  s_ref[...] = s
  d_ref[...] = d


def _hfin_call(hprev, hnew, st, g, b, wp, bp):
  return pl.pallas_call(
      _hfin_body,
      grid=(NSTEPS,),
      in_specs=[
          pl.BlockSpec((BN_NODE, DP), lambda i: (i, 0)),
          pl.BlockSpec((BN_NODE, DP), lambda i: (i, 0)),
          pl.BlockSpec((2, DP), lambda i: (0, 0)),
          pl.BlockSpec((1, DP), lambda i: (0, 0)),
          pl.BlockSpec((1, DP), lambda i: (0, 0)),
          pl.BlockSpec((DP, 4 * DP), lambda i: (0, 0)),
          pl.BlockSpec((1, 4 * DP), lambda i: (0, 0)),
      ],
      out_specs=[
          pl.BlockSpec((BN_NODE, DP), lambda i: (i, 0)),
          pl.BlockSpec((BN_NODE, DP), lambda i: (i, 0)),
          pl.BlockSpec((G, BN_NODE, 32), lambda i: (0, i, 0)),
          pl.BlockSpec((G, BN_NODE, 16), lambda i: (0, i, 0)),
      ],
      out_shape=[
          jax.ShapeDtypeStruct((N, DP), jnp.float32),
          jax.ShapeDtypeStruct((N, DP), jnp.float32),
          jax.ShapeDtypeStruct((G, N, 32), jnp.float32),
          jax.ShapeDtypeStruct((G, N, 16), jnp.float32),
      ],
  )(hprev, hnew, st, g, b, wp, bp)


BE = 2048
ESTEPS = EPAD // BE


def _ce2_body(t_ref, ef_ref, st_ref, werow_ref, be_ref, g1_ref, b1_ref,
              cw_ref, cb_ref, out_ref):
  parts = []
  for g in range(G):
    sums = jnp.sum(st_ref[:, :, g, 0:16], axis=(0, 1))
    sqs = jnp.sum(st_ref[:, :, g, 16:32], axis=(0, 1))
    m = sums / E
    v = sqs / E - m * m
    sl = slice(16 * g, 16 * (g + 1))
    tg = t_ref[:, sl]
    bn = g1_ref[0:1, sl] * (tg - m[None, :]) * lax.rsqrt(v + 1e-5)[None, :] \
        + b1_ref[0:1, sl]
    e0g = ef_ref[...] * werow_ref[0:1, sl] + be_ref[0:1, sl]
    parts.append(e0g + jnp.maximum(bn, 0.0))
  e1 = jnp.concatenate(parts, axis=1)
  out_ref[...] = jnp.dot(e1, cw_ref[...],
                         preferred_element_type=jnp.float32) + cb_ref[...]


def _ce2_call(t, ef, st, werow, be, g1, b1, cw, cb):
  return pl.pallas_call(
      _ce2_body,
      grid=(ESTEPS,),
      in_specs=[
          pl.BlockSpec((BE, DP), lambda i: (i, 0)),
          pl.BlockSpec((BE, 1), lambda i: (i, 0)),
          pl.BlockSpec((2, 16, G, 32), lambda i: (0, 0, 0, 0)),
          pl.BlockSpec((1, DP), lambda i: (0, 0)),
          pl.BlockSpec((1, DP), lambda i: (0, 0)),
          pl.BlockSpec((1, DP), lambda i: (0, 0)),
          pl.BlockSpec((1, DP), lambda i: (0, 0)),
          pl.BlockSpec((DP, DP), lambda i: (0, 0)),
          pl.BlockSpec((1, DP), lambda i: (0, 0)),
      ],
      out_specs=pl.BlockSpec((BE, DP), lambda i: (i, 0)),
      out_shape=jax.ShapeDtypeStruct((EPAD, DP), jnp.float32),
  )(t, ef, st, werow, be, g1, b1, cw, cb)


def _final_body(hprev_ref, hnew_ref, st_ref, g_ref, b_ref, out_ref, acc_ref):
  i = pl.program_id(0)
  m = st_ref[0:1, :] / N
  v = st_ref[1:2, :] / N - m * m
  hn = g_ref[...] * (hnew_ref[...] - m) * lax.rsqrt(v + 1e-5) + b_ref[...]
  h2 = hprev_ref[...] + jnp.maximum(hn, 0.0)

  @pl.when(i == 0)
  def _():
    acc_ref[...] = jnp.zeros_like(acc_ref)

  acc_ref[...] += jnp.sum(h2, axis=0, keepdims=True)

  @pl.when(i == NSTEPS - 1)
  def _():
    out_ref[...] = acc_ref[...] / N


def _final_call(hprev, hnew, st, g, b):
  return pl.pallas_call(
      _final_body,
      grid=(NSTEPS,),
      in_specs=[
          pl.BlockSpec((BN_NODE, DP), lambda i: (i, 0)),
          pl.BlockSpec((BN_NODE, DP), lambda i: (i, 0)),
          pl.BlockSpec((2, DP), lambda i: (0, 0)),
          pl.BlockSpec((1, DP), lambda i: (0, 0)),
          pl.BlockSpec((1, DP), lambda i: (0, 0)),
      ],
      out_specs=pl.BlockSpec((1, DP), lambda i: (0, 0)),
      out_shape=jax.ShapeDtypeStruct((1, DP), jnp.float32),
      scratch_shapes=[pltpu.VMEM((1, DP), jnp.float32)],
  )(hprev, hnew, st, g, b)


@functools.cache
def _sc_mesh():
  return plsc.VectorSubcoreMesh(core_axis_name="c", subcore_axis_name="s",
                                num_cores=2, num_subcores=16)


def _sc_layer1(sidx, didx, efv, snv, stab, dtab, u, w, zrow):
  out_type = (
      jax.ShapeDtypeStruct((2, G, NT, 32), jnp.float32),
      jax.ShapeDtypeStruct((EPAD, DP), jnp.float32),
      jax.ShapeDtypeStruct((2, 16, G, 32), jnp.float32),
  )
  scratch = [
      pltpu.VMEM_SHARED((NT, 32), jnp.float32),
      pltpu.VMEM((BK,), jnp.int32),
      pltpu.VMEM((BK,), jnp.int32),
      pltpu.VMEM((BK,), jnp.float32),
      pltpu.VMEM((BK,), jnp.float32),
      pltpu.VMEM((BK,), jnp.int32),
      pltpu.VMEM((BK,), jnp.int32),
      pltpu.VMEM((BK, 32), jnp.float32),
      pltpu.VMEM((BK, 16), jnp.float32),
      pltpu.VMEM((BK, 32), jnp.float32),
      pltpu.VMEM((BK, 16), jnp.float32),
      pltpu.VMEM((DP,), jnp.float32),
      pltpu.VMEM((DP,), jnp.float32),
      pltpu.VMEM((32,), jnp.float32),
  ]

  @functools.partial(pl.kernel, out_type=out_type, mesh=_sc_mesh(),
                     scratch_types=scratch,
                     compiler_params=pltpu.CompilerParams(
                         use_tc_tiling_on_sc=False,
                         needs_layout_passes=False))
  def body(sidx_h, didx_h, ef_h, sn_h, stab_h, dtab_h, u_h, w_h, z_h,
           nd_h, t_h, st_h,
           acc, sidx_b, didx_b, ef_b, sn_b, sgi_v, gdi_v, Sv, Dv, Pv, Tv,
           uv, wv, stbuf):
    c = lax.axis_index("c")
    s = lax.axis_index("s")
    wid = c * 16 + s
    ebase = wid * EPT
    pltpu.sync_copy(u_h, uv)
    pltpu.sync_copy(w_h, wv)

    for g in range(G):
      pltpu.sync_copy(z_h, acc.at[pl.ds(s * RPS, RPS)])
      plsc.subcore_barrier()
      ug = uv[pl.ds(16 * g, 16)]
      wg = wv[pl.ds(16 * g, 16)]

      def blk(b, carry):
        pltpu.sync_copy(sidx_h.at[wid, b], sidx_b)
        pltpu.sync_copy(didx_h.at[wid, b], didx_b)
        pltpu.sync_copy(ef_h.at[wid, b], ef_b)
        pltpu.sync_copy(sn_h.at[wid, b], sn_b)
        for kk in range(BK // 16):
          sgi_v[pl.ds(kk * 16, 16)] = sidx_b[pl.ds(kk * 16, 16)] + g * N
          gdi_v[pl.ds(kk * 16, 16)] = didx_b[pl.ds(kk * 16, 16)] + g * NT
        pltpu.sync_copy(stab_h.at[sgi_v], Sv)
        pltpu.sync_copy(dtab_h.at[gdi_v], Dv)

        def edge(i, ec):
          ssum, ssq = ec
          bh = Sv[i, pl.ds(0, 16)]
          dh = Sv[i, pl.ds(16, 16)]
          eh = Dv[i, pl.ds(0, 16)]
          ifull = jnp.full((16,), i, jnp.int32)
          efb = plsc.load_gather(ef_b, [ifull])
          ce = efb * ug + wg
          en = ce + dh + eh
          sig = 1.0 / (1.0 + jnp.exp(-en))
          Pv[i, pl.ds(0, 16)] = sig * bh
          Pv[i, pl.ds(16, 16)] = sig
          snb = plsc.load_gather(sn_b, [ifull])
          t = en * snb
          Tv[i, pl.ds(0, 16)] = t
          return ssum + t, ssq + t * t

        carry = lax.fori_loop(0, BK, edge, carry)
        pltpu.sync_copy(Pv, acc.at[didx_b], add=True)
        pltpu.sync_copy(
            Tv, t_h.at[pl.ds(ebase + b * BK, BK), pl.ds(16 * g, 16)])
        return carry

      z16 = jnp.zeros((16,), jnp.float32)
      ssum, ssq = lax.fori_loop(0, NBLK, blk, (z16, z16))
      stbuf[pl.ds(0, 16)] = ssum
      stbuf[pl.ds(16, 16)] = ssq
      pltpu.sync_copy(stbuf, st_h.at[c, s, g])
      plsc.subcore_barrier()
      pltpu.sync_copy(acc.at[pl.ds(s * RPS, RPS)],
                      nd_h.at[c, g, pl.ds(s * RPS, RPS)])

  return body(sidx, didx, efv, snv, stab, dtab, u, w, zrow)


def _sc_layer2(sidx, didx, ce, stab, dtab, zrow):
  out_type = jax.ShapeDtypeStruct((2, G, NT, 32), jnp.float32)
  scratch = [
      pltpu.VMEM_SHARED((NT, 32), jnp.float32),
      pltpu.VMEM((BK,), jnp.int32),
      pltpu.VMEM((BK,), jnp.int32),
      pltpu.VMEM((BK,), jnp.int32),
      pltpu.VMEM((BK,), jnp.int32),
      pltpu.VMEM((BK, 32), jnp.float32),
      pltpu.VMEM((BK, 16), jnp.float32),
      pltpu.VMEM((BK, 16), jnp.float32),
      pltpu.VMEM((BK, 32), jnp.float32),
  ]

  @functools.partial(pl.kernel, out_type=out_type, mesh=_sc_mesh(),
                     scratch_types=scratch,
                     compiler_params=pltpu.CompilerParams(
                         use_tc_tiling_on_sc=False,
                         needs_layout_passes=False))
  def body(sidx_h, didx_h, ce_h, stab_h, dtab_h, z_h, nd_h,
           acc, sidx_b, didx_b, sgi_v, gdi_v, Sv, Dv, Cv, Pv):
    c = lax.axis_index("c")
    s = lax.axis_index("s")
    wid = c * 16 + s
    ebase = wid * EPT

    for g in range(G):
      pltpu.sync_copy(z_h, acc.at[pl.ds(s * RPS, RPS)])
      plsc.subcore_barrier()

      def blk(b, carry):
        pltpu.sync_copy(sidx_h.at[wid, b], sidx_b)
        pltpu.sync_copy(didx_h.at[wid, b], didx_b)
        for kk in range(BK // 16):
          sgi_v[pl.ds(kk * 16, 16)] = sidx_b[pl.ds(kk * 16, 16)] + g * N
          gdi_v[pl.ds(kk * 16, 16)] = didx_b[pl.ds(kk * 16, 16)] + g * NT
        pltpu.sync_copy(stab_h.at[sgi_v], Sv)
        pltpu.sync_copy(dtab_h.at[gdi_v], Dv)
        pltpu.sync_copy(
            ce_h.at[pl.ds(ebase + b * BK, BK), pl.ds(16 * g, 16)], Cv)

        def edge(i, ec):
          bh = Sv[i, pl.ds(0, 16)]
          dh = Sv[i, pl.ds(16, 16)]
          eh = Dv[i, pl.ds(0, 16)]
          en = Cv[i, pl.ds(0, 16)] + dh + eh
          sig = 1.0 / (1.0 + jnp.exp(-en))
          Pv[i, pl.ds(0, 16)] = sig * bh
          Pv[i, pl.ds(16, 16)] = sig
          return ec

        lax.fori_loop(0, BK, edge, 0)
        pltpu.sync_copy(Pv, acc.at[didx_b], add=True)
        return carry

      lax.fori_loop(0, NBLK, blk, 0)
      plsc.subcore_barrier()
      pltpu.sync_copy(acc.at[pl.ds(s * RPS, RPS)],
                      nd_h.at[c, g, pl.ds(s * RPS, RPS)])

  return body(sidx, didx, ce, stab, dtab, zrow)


def _padcol(a, width):
  return jnp.pad(a, ((0, 0), (0, width - a.shape[1])))


def _padvec(v, width):
  return jnp.pad(v, (0, width - v.shape[0]))


def _run(edge_index, nodes_feat, edges_feat, snorm_n, snorm_e, params):
  f32 = jnp.float32
  src = edge_index[0]
  dst = edge_index[1]

  whp = _padcol(params["Wh"], DP)
  bhp = _padvec(params["bh"], DP)[None, :]
  lp = params["layers"]

  def pack_layer(p):
    wp = jnp.concatenate(
        [_padcol(p[k + "_W"], DP) for k in "ABDE"], axis=1)
    wp = jnp.pad(wp, ((0, DP - D), (0, 0)))
    bp = jnp.concatenate([_padvec(p[k + "_b"], DP) for k in "ABDE"])[None, :]
    return wp, bp

  wp1, bp1 = pack_layer(lp[0])
  wp2, bp2 = pack_layer(lp[1])

  we_row = _padvec(params["We"][0], DP)[None, :]
  bep = _padvec(params["be"], DP)[None, :]
  u1 = _padvec(params["We"][0] @ lp[0]["C_W"], DP)
  w1 = _padvec(params["be"] @ lp[0]["C_W"] + lp[0]["C_b"], DP)
  c2w = jnp.pad(_padcol(lp[1]["C_W"], DP), ((0, DP - D), (0, 0)))
  c2b = _padvec(lp[1]["C_b"], DP)[None, :]
  bnh1_g = _padvec(lp[0]["bnh_g"], DP)[None, :]
  bnh1_b = _padvec(lp[0]["bnh_b"], DP)[None, :]
  bne1_g = _padvec(lp[0]["bne_g"], DP)[None, :]
  bne1_b = _padvec(lp[0]["bne_b"], DP)[None, :]
  bnh2_g = _padvec(lp[1]["bnh_g"], DP)[None, :]
  bnh2_b = _padvec(lp[1]["bnh_b"], DP)[None, :]

  npad = EPAD - E
  srcp = jnp.concatenate([src, jnp.zeros((npad,), jnp.int32)]).reshape(
      NW, NBLK, BK)
  dstp = jnp.concatenate([dst, jnp.full((npad,), N, jnp.int32)]).reshape(
      NW, NBLK, BK)
  efp = jnp.concatenate([edges_feat[:, 0],
                         jnp.zeros((npad,), f32)]).reshape(NW, NBLK, BK)
  snp = jnp.concatenate([snorm_e[:, 0],
                         jnp.zeros((npad,), f32)]).reshape(NW, NBLK, BK)
  ef2d = jnp.concatenate([edges_feat[:, 0],
                          jnp.zeros((npad,), f32)])[:, None]
  zrow = jnp.zeros((RPS, 32), f32)

  h0, a1, s1, d1 = _entry_call(nodes_feat, whp, bhp, wp1, bp1)
  stab1 = s1.reshape(G * N, 32)
  dtab1 = jnp.pad(d1, ((0, 0), (0, NT - N), (0, 0))).reshape(G * NT, 16)
  nd1, t1, st1 = _sc_layer1(srcp, dstp, efp, snp, stab1, dtab1, u1, w1, zrow)
  hnew1, hst1 = _hnew_call(a1, nd1, snorm_n)
  h1, a2, s2, d2 = _hfin_call(h0, hnew1, hst1, bnh1_g, bnh1_b, wp2, bp2)

  ce2 = _ce2_call(t1, ef2d, st1, we_row, bep, bne1_g, bne1_b, c2w, c2b)
  stab2 = s2.reshape(G * N, 32)
  dtab2 = jnp.pad(d2, ((0, 0), (0, NT - N), (0, 0))).reshape(G * NT, 16)
  nd2 = _sc_layer2(srcp, dstp, ce2, stab2, dtab2, zrow)
  hnew2, hst2 = _hnew_call(a2, nd2, snorm_n)
  out = _final_call(h1, hnew2, hst2, bnh2_g, bnh2_b)
  return out[:, :D]


_run_jit = jax.jit(_run)


def kernel(edge_index, nodes_feat, edges_feat, nodes_num_norm_sqrt,
           edges_num_norm_sqrt, params):
  return _run_jit(edge_index, nodes_feat, edges_feat, nodes_num_norm_sqrt,
              edges_num_norm_sqrt, params)


# speedup vs baseline: 3.9018x; 3.4019x over previous
import functools

import jax
import jax.numpy as jnp
from jax import lax
from jax.experimental import pallas as pl
from jax.experimental.pallas import tpu as pltpu
from jax.experimental.pallas import tpu_sc as plsc

N = 50000
E = 800000
IN_DIM = 64
D = 70
DP = 80
G = 5
NW = 32
BK = 128
NBLK = 196
EPT = NBLK * BK
EPAD = NW * EPT
NT = 50016
RPS = NT // 16


BN_NODE = 2000
NSTEPS = N // BN_NODE


def _pack_tables(tabs):
  Bh = tabs[:, DP:2 * DP]
  Dh = tabs[:, 2 * DP:3 * DP]
  Eh = tabs[:, 3 * DP:4 * DP]
  src = jnp.stack([
      jnp.concatenate([Bh[:, 16 * g:16 * (g + 1)], Dh[:, 16 * g:16 * (g + 1)]],
                      axis=1) for g in range(G)], axis=0)
  dst = jnp.stack([Eh[:, 16 * g:16 * (g + 1)] for g in range(G)], axis=0)
  return src, dst


def _entry_body(x_ref, wh_ref, bh_ref, wp_ref, bp_ref,
                h0_ref, a_ref, s_ref, d_ref):
  h0 = jnp.dot(x_ref[...], wh_ref[...],
               preferred_element_type=jnp.float32) + bh_ref[...]
  tabs = jnp.dot(h0, wp_ref[...],
                 preferred_element_type=jnp.float32) + bp_ref[...]
  h0_ref[...] = h0
  a_ref[...] = tabs[:, :DP]
  s, d = _pack_tables(tabs)
  s_ref[...] = s
  d_ref[...] = d


def _entry_call(x, wh, bh, wp, bp):
  return pl.pallas_call(
      _entry_body,
      grid=(NSTEPS,),
      in_specs=[
          pl.BlockSpec((BN_NODE, IN_DIM), lambda i: (i, 0)),
          pl.BlockSpec((IN_DIM, DP), lambda i: (0, 0)),
          pl.BlockSpec((1, DP), lambda i: (0, 0)),
          pl.BlockSpec((DP, 4 * DP), lambda i: (0, 0)),
          pl.BlockSpec((1, 4 * DP), lambda i: (0, 0)),
      ],
      out_specs=[
          pl.BlockSpec((BN_NODE, DP), lambda i: (i, 0)),
          pl.BlockSpec((BN_NODE, DP), lambda i: (i, 0)),
          pl.BlockSpec((G, BN_NODE, 32), lambda i: (0, i, 0)),
          pl.BlockSpec((G, BN_NODE, 16), lambda i: (0, i, 0)),
      ],
      out_shape=[
          jax.ShapeDtypeStruct((N, DP), jnp.float32),
          jax.ShapeDtypeStruct((N, DP), jnp.float32),
          jax.ShapeDtypeStruct((G, N, 32), jnp.float32),
          jax.ShapeDtypeStruct((G, N, 16), jnp.float32),
      ],
  )(x, wh, bh, wp, bp)


def _hnew_body(a_ref, nd_ref, snn_ref, est_ref, h_ref, st_ref, ste_ref,
               acc_ref):
  i = pl.program_id(0)
  num = jnp.concatenate(
      [nd_ref[0, g, :, 0:16] + nd_ref[1, g, :, 0:16] for g in range(G)],
      axis=1)
  den = jnp.concatenate(
      [nd_ref[0, g, :, 16:32] + nd_ref[1, g, :, 16:32] for g in range(G)],
      axis=1)
  hnew = (a_ref[...] + num / (den + 1e-6)) * snn_ref[...]
  h_ref[...] = hnew

  @pl.when(i == 0)
  def _():
    acc_ref[...] = jnp.zeros_like(acc_ref)

  acc_ref[0:1, :] += jnp.sum(hnew, axis=0, keepdims=True)
  acc_ref[1:2, :] += jnp.sum(hnew * hnew, axis=0, keepdims=True)

  @pl.when(i == NSTEPS - 1)
  def _():
    st_ref[...] = acc_ref[...]
    sums = jnp.concatenate(
        [jnp.sum(est_ref[:, :, g, 0:16], axis=(0, 1)) for g in range(G)])
    sqs = jnp.concatenate(
        [jnp.sum(est_ref[:, :, g, 16:32], axis=(0, 1)) for g in range(G)])
    ste_ref[...] = jnp.stack([sums, sqs], axis=0)


def _hnew_call(a, nd, snn, est):
  return pl.pallas_call(
      _hnew_body,
      grid=(NSTEPS,),
      in_specs=[
          pl.BlockSpec((BN_NODE, DP), lambda i: (i, 0)),
          pl.BlockSpec((2, G, BN_NODE, 32), lambda i: (0, 0, i, 0)),
          pl.BlockSpec((BN_NODE, 1), lambda i: (i, 0)),
          pl.BlockSpec((2, 16, G, 32), lambda i: (0, 0, 0, 0)),
      ],
      out_specs=[
          pl.BlockSpec((BN_NODE, DP), lambda i: (i, 0)),
          pl.BlockSpec((2, DP), lambda i: (0, 0)),
          pl.BlockSpec((2, DP), lambda i: (0, 0)),
      ],
      out_shape=[
          jax.ShapeDtypeStruct((N, DP), jnp.float32),
          jax.ShapeDtypeStruct((2, DP), jnp.float32),
          jax.ShapeDtypeStruct((2, DP), jnp.float32),
      ],
      scratch_shapes=[pltpu.VMEM((2, DP), jnp.float32)],
  )(a, nd, snn, est)


def _hfin_body(hprev_ref, hnew_ref, st_ref, g_ref, b_ref, wp_ref, bp_ref,
               h_ref, a_ref, s_ref, d_ref):
  m = st_ref[0:1, :] / N
  v = st_ref[1:2, :] / N - m * m
  hn = g_ref[...] * (hnew_ref[...] - m) * lax.rsqrt(v + 1e-5) + b_ref[...]
  h1 = hprev_ref[...] + jnp.maximum(hn, 0.0)
  h_ref[...] = h1
  tabs = jnp.dot(h1, wp_ref[...],
                 preferred_element_type=jnp.float32) + bp_ref[...]
  a_ref[...] = tabs[:, :DP]
  s, d = _pack_tables(tabs)
  s_ref[...] = s
  d_ref[...] = d


def _hfin_call(hprev, hnew, st, g, b, wp, bp):
  return pl.pallas_call(
      _hfin_body,
      grid=(NSTEPS,),
      in_specs=[
          pl.BlockSpec((BN_NODE, DP), lambda i: (i, 0)),
          pl.BlockSpec((BN_NODE, DP), lambda i: (i, 0)),
          pl.BlockSpec((2, DP), lambda i: (0, 0)),
          pl.BlockSpec((1, DP), lambda i: (0, 0)),
          pl.BlockSpec((1, DP), lambda i: (0, 0)),
          pl.BlockSpec((DP, 4 * DP), lambda i: (0, 0)),
          pl.BlockSpec((1, 4 * DP), lambda i: (0, 0)),
      ],
      out_specs=[
          pl.BlockSpec((BN_NODE, DP), lambda i: (i, 0)),
          pl.BlockSpec((BN_NODE, DP), lambda i: (i, 0)),
          pl.BlockSpec((G, BN_NODE, 32), lambda i: (0, i, 0)),
          pl.BlockSpec((G, BN_NODE, 16), lambda i: (0, i, 0)),
      ],
      out_shape=[
          jax.ShapeDtypeStruct((N, DP), jnp.float32),
          jax.ShapeDtypeStruct((N, DP), jnp.float32),
          jax.ShapeDtypeStruct((G, N, 32), jnp.float32),
          jax.ShapeDtypeStruct((G, N, 16), jnp.float32),
      ],
  )(hprev, hnew, st, g, b, wp, bp)


BE = 2048
ESTEPS = EPAD // BE


def _ce2_body(t_ref, ef_ref, ste_ref, werow_ref, be_ref, g1_ref, b1_ref,
              cw_ref, cb_ref, out_ref):
  m = ste_ref[0:1, :] / E
  v = ste_ref[1:2, :] / E - m * m
  bn = g1_ref[...] * (t_ref[...] - m) * lax.rsqrt(v + 1e-5) + b1_ref[...]
  e1 = ef_ref[...] * werow_ref[...] + be_ref[...] + jnp.maximum(bn, 0.0)
  out_ref[...] = jnp.dot(e1, cw_ref[...],
                         preferred_element_type=jnp.float32) + cb_ref[...]


def _ce2_call(t, ef, ste, werow, be, g1, b1, cw, cb):
  return pl.pallas_call(
      _ce2_body,
      grid=(ESTEPS,),
      in_specs=[
          pl.BlockSpec((BE, DP), lambda i: (i, 0)),
          pl.BlockSpec((BE, 1), lambda i: (i, 0)),
          pl.BlockSpec((2, DP), lambda i: (0, 0)),
          pl.BlockSpec((1, DP), lambda i: (0, 0)),
          pl.BlockSpec((1, DP), lambda i: (0, 0)),
          pl.BlockSpec((1, DP), lambda i: (0, 0)),
          pl.BlockSpec((1, DP), lambda i: (0, 0)),
          pl.BlockSpec((DP, DP), lambda i: (0, 0)),
          pl.BlockSpec((1, DP), lambda i: (0, 0)),
      ],
      out_specs=pl.BlockSpec((BE, DP), lambda i: (i, 0)),
      out_shape=jax.ShapeDtypeStruct((EPAD, DP), jnp.float32),
  )(t, ef, ste, werow, be, g1, b1, cw, cb)


def _final_body(hprev_ref, hnew_ref, st_ref, g_ref, b_ref, out_ref, acc_ref):
  i = pl.program_id(0)
  m = st_ref[0:1, :] / N
  v = st_ref[1:2, :] / N - m * m
  hn = g_ref[...] * (hnew_ref[...] - m) * lax.rsqrt(v + 1e-5) + b_ref[...]
  h2 = hprev_ref[...] + jnp.maximum(hn, 0.0)

  @pl.when(i == 0)
  def _():
    acc_ref[...] = jnp.zeros_like(acc_ref)

  acc_ref[...] += jnp.sum(h2, axis=0, keepdims=True)

  @pl.when(i == NSTEPS - 1)
  def _():
    out_ref[...] = acc_ref[...] / N


def _final_call(hprev, hnew, st, g, b):
  return pl.pallas_call(
      _final_body,
      grid=(NSTEPS,),
      in_specs=[
          pl.BlockSpec((BN_NODE, DP), lambda i: (i, 0)),
          pl.BlockSpec((BN_NODE, DP), lambda i: (i, 0)),
          pl.BlockSpec((2, DP), lambda i: (0, 0)),
          pl.BlockSpec((1, DP), lambda i: (0, 0)),
          pl.BlockSpec((1, DP), lambda i: (0, 0)),
      ],
      out_specs=pl.BlockSpec((1, DP), lambda i: (0, 0)),
      out_shape=jax.ShapeDtypeStruct((1, DP), jnp.float32),
      scratch_shapes=[pltpu.VMEM((1, DP), jnp.float32)],
  )(hprev, hnew, st, g, b)


@functools.cache
def _sc_mesh():
  return plsc.VectorSubcoreMesh(core_axis_name="c", subcore_axis_name="s",
                                num_cores=2, num_subcores=16)


def _sc_layer1(idx4, stab, dtab, u, zrow):
  out_type = (
      jax.ShapeDtypeStruct((2, G, NT, 32), jnp.float32),
      jax.ShapeDtypeStruct((EPAD, DP), jnp.float32),
      jax.ShapeDtypeStruct((2, 16, G, 32), jnp.float32),
  )
  scratch = [
      pltpu.VMEM_SHARED((NT, 32), jnp.float32),
      [pltpu.VMEM((4, BK), jnp.int32) for _ in range(2)],
      [pltpu.VMEM((BK,), jnp.int32) for _ in range(2)],
      [pltpu.VMEM((BK,), jnp.int32) for _ in range(2)],
      [pltpu.VMEM((BK,), jnp.int32) for _ in range(2)],
      [pltpu.VMEM((BK,), jnp.float32) for _ in range(2)],
      [pltpu.VMEM((BK,), jnp.float32) for _ in range(2)],
      [pltpu.VMEM((BK, 32), jnp.float32) for _ in range(2)],
      [pltpu.VMEM((BK, 16), jnp.float32) for _ in range(2)],
      [pltpu.VMEM((BK, 32), jnp.float32) for _ in range(2)],
      [pltpu.VMEM((BK, 16), jnp.float32) for _ in range(2)],
      pltpu.VMEM((DP,), jnp.float32),
      pltpu.VMEM((32,), jnp.float32),
      [pltpu.SemaphoreType.DMA for _ in range(2)],
      [pltpu.SemaphoreType.DMA for _ in range(2)],
      [pltpu.SemaphoreType.DMA for _ in range(2)],
      [pltpu.SemaphoreType.DMA for _ in range(2)],
  ]

  @functools.partial(pl.kernel, out_type=out_type, mesh=_sc_mesh(),
                     scratch_types=scratch,
                     compiler_params=pltpu.CompilerParams(
                         use_tc_tiling_on_sc=False,
                         needs_layout_passes=False))
  def body(idx4_h, stab_h, dtab_h, u_h, z_h,
           nd_h, t_h, st_h,
           acc, IDX, SGI, GDI, DSTS, EFS, SNS, SV, DV, PV, TV,
           uv, stbuf, sl, sg, ss, st):
    c = lax.axis_index("c")
    s = lax.axis_index("s")
    wid = c * 16 + s
    ebase = wid * EPT
    pltpu.sync_copy(u_h, uv)
    z16 = jnp.zeros((16,), jnp.float32)

    for g in range(G):
      pltpu.sync_copy(z_h, acc.at[pl.ds(s * RPS, RPS)])
      plsc.subcore_barrier()
      ug = uv[pl.ds(16 * g, 16)]

      def t_dst(bb):
        return t_h.at[pl.ds(ebase + bb * BK, BK), pl.ds(16 * g, 16)]

      def body2(j, carry):
        for par in range(2):
          bb = 2 * j + par
          p_i = (par + 1) % 2
          p_c = par

          @pl.when(bb < NBLK)
          def _():
            pltpu.async_copy(idx4_h.at[wid, bb], IDX[p_c], sl[p_c])

          @pl.when(jnp.logical_and(bb >= 3, bb < NBLK + 3))
          def _():
            pltpu.make_async_copy(PV[p_i], acc.at[DSTS[p_i]], ss[p_i]).wait()
            pltpu.make_async_copy(TV[p_i], t_dst(bb - 3), st[p_i]).wait()

          @pl.when(jnp.logical_and(bb >= 1, bb < NBLK + 1))
          def _():
            pltpu.make_async_copy(
                idx4_h.at[wid, bb - 1], IDX[p_i], sl[p_i]).wait()
            for kk in range(BK // 16):
              sl16 = pl.ds(kk * 16, 16)
              srow = IDX[p_i][0, sl16]
              drow = IDX[p_i][1, sl16]
              SGI[p_i][sl16] = srow + g * N
              GDI[p_i][sl16] = drow + g * NT
              DSTS[p_i][sl16] = drow
              EFS[p_i][sl16] = plsc.bitcast(IDX[p_i][2, sl16], jnp.float32)
              SNS[p_i][sl16] = plsc.bitcast(IDX[p_i][3, sl16], jnp.float32)
            pltpu.async_copy(stab_h.at[SGI[p_i]], SV[p_i], sg[p_i])
            pltpu.async_copy(dtab_h.at[GDI[p_i]], DV[p_i], sg[p_i])

          def c_stage(ec):
            pltpu.make_async_copy(stab_h.at[SGI[p_c]], SV[p_c], sg[p_c]).wait()
            pltpu.make_async_copy(dtab_h.at[GDI[p_c]], DV[p_c], sg[p_c]).wait()

            @plsc.parallel_loop(0, BK, unroll=4, carry=ec)
            def edge_loop(i, ec2):
              ssum, ssq = ec2
              bh = SV[p_c][i, pl.ds(0, 16)]
              dh = SV[p_c][i, pl.ds(16, 16)]
              eh = DV[p_c][i, pl.ds(0, 16)]
              ifull = jnp.full((16,), i, jnp.int32)
              efb = plsc.load_gather(EFS[p_c], [ifull])
              snb = plsc.load_gather(SNS[p_c], [ifull])
              en = efb * ug + dh + eh
              sig = 1.0 / (1.0 + jnp.exp(-en))
              PV[p_c][i, pl.ds(0, 16)] = sig * bh
              PV[p_c][i, pl.ds(16, 16)] = sig
              t = en * snb
              TV[p_c][i, pl.ds(0, 16)] = t
              return ssum + t, ssq + t * t

            ec = edge_loop
            pltpu.async_copy(PV[p_c], acc.at[DSTS[p_c]], ss[p_c], add=True)
            pltpu.async_copy(TV[p_c], t_dst(bb - 2), st[p_c])
            return ec

          carry = lax.cond(
              jnp.logical_and(bb >= 2, bb < NBLK + 2),
              c_stage, lambda ec: ec, carry)
        return carry

      ssum, ssq = lax.fori_loop(0, (NBLK + 4) // 2, body2, (z16, z16))
      stbuf[pl.ds(0, 16)] = ssum
      stbuf[pl.ds(16, 16)] = ssq
      pltpu.sync_copy(stbuf, st_h.at[c, s, g])
      plsc.subcore_barrier()
      pltpu.sync_copy(acc.at[pl.ds(s * RPS, RPS)],
                      nd_h.at[c, g, pl.ds(s * RPS, RPS)])

  return body(idx4, stab, dtab, u, zrow)


def _sc_layer2(idx2, ce, stab, dtab, zrow):
  out_type = jax.ShapeDtypeStruct((2, G, NT, 32), jnp.float32)
  scratch = [
      pltpu.VMEM_SHARED((NT, 32), jnp.float32),
      [pltpu.VMEM((2, BK), jnp.int32) for _ in range(2)],
      [pltpu.VMEM((BK,), jnp.int32) for _ in range(2)],
      [pltpu.VMEM((BK,), jnp.int32) for _ in range(2)],
      [pltpu.VMEM((BK,), jnp.int32) for _ in range(2)],
      [pltpu.VMEM((BK, 32), jnp.float32) for _ in range(2)],
      [pltpu.VMEM((BK, 16), jnp.float32) for _ in range(2)],
      [pltpu.VMEM((BK, 16), jnp.float32) for _ in range(2)],
      [pltpu.VMEM((BK, 32), jnp.float32) for _ in range(2)],
      [pltpu.SemaphoreType.DMA for _ in range(2)],
      [pltpu.SemaphoreType.DMA for _ in range(2)],
      [pltpu.SemaphoreType.DMA for _ in range(2)],
  ]

  @functools.partial(pl.kernel, out_type=out_type, mesh=_sc_mesh(),
                     scratch_types=scratch,
                     compiler_params=pltpu.CompilerParams(
                         use_tc_tiling_on_sc=False,
                         needs_layout_passes=False))
  def body(idx2_h, ce_h, stab_h, dtab_h, z_h, nd_h,
           acc, IDX, SGI, GDI, DSTS, SV, DV, CV, PV, sl, sg, ss):
    c = lax.axis_index("c")
    s = lax.axis_index("s")
    wid = c * 16 + s
    ebase = wid * EPT

    for g in range(G):
      pltpu.sync_copy(z_h, acc.at[pl.ds(s * RPS, RPS)])
      plsc.subcore_barrier()

      def ce_src(bb):
        return ce_h.at[pl.ds(ebase + bb * BK, BK), pl.ds(16 * g, 16)]

      def body2(j, carry):
        for par in range(2):
          bb = 2 * j + par
          p_i = (par + 1) % 2
          p_c = par

          @pl.when(bb < NBLK)
          def _():
            pltpu.async_copy(idx2_h.at[wid, bb], IDX[p_c], sl[p_c])

          @pl.when(jnp.logical_and(bb >= 3, bb < NBLK + 3))
          def _():
            pltpu.make_async_copy(PV[p_i], acc.at[DSTS[p_i]], ss[p_i]).wait()

          @pl.when(jnp.logical_and(bb >= 1, bb < NBLK + 1))
          def _():
            pltpu.make_async_copy(
                idx2_h.at[wid, bb - 1], IDX[p_i], sl[p_i]).wait()
            for kk in range(BK // 16):
              sl16 = pl.ds(kk * 16, 16)
              srow = IDX[p_i][0, sl16]
              drow = IDX[p_i][1, sl16]
              SGI[p_i][sl16] = srow + g * N
              GDI[p_i][sl16] = drow + g * NT
              DSTS[p_i][sl16] = drow
            pltpu.async_copy(stab_h.at[SGI[p_i]], SV[p_i], sg[p_i])
            pltpu.async_copy(dtab_h.at[GDI[p_i]], DV[p_i], sg[p_i])
            pltpu.async_copy(ce_src(bb - 1), CV[p_i], sg[p_i])

          @pl.when(jnp.logical_and(bb >= 2, bb < NBLK + 2))
          def _():
            pltpu.make_async_copy(stab_h.at[SGI[p_c]], SV[p_c], sg[p_c]).wait()
            pltpu.make_async_copy(dtab_h.at[GDI[p_c]], DV[p_c], sg[p_c]).wait()
            pltpu.make_async_copy(ce_src(bb - 2), CV[p_c], sg[p_c]).wait()

            @plsc.parallel_loop(0, BK, unroll=4)
            def edge_loop(i):
              bh = SV[p_c][i, pl.ds(0, 16)]
              dh = SV[p_c][i, pl.ds(16, 16)]
              eh = DV[p_c][i, pl.ds(0, 16)]
              en = CV[p_c][i, pl.ds(0, 16)] + dh + eh
              sig = 1.0 / (1.0 + jnp.exp(-en))
              PV[p_c][i, pl.ds(0, 16)] = sig * bh
              PV[p_c][i, pl.ds(16, 16)] = sig

            pltpu.async_copy(PV[p_c], acc.at[DSTS[p_c]], ss[p_c], add=True)
        return carry

      lax.fori_loop(0, (NBLK + 4) // 2, body2, 0)
      plsc.subcore_barrier()
      pltpu.sync_copy(acc.at[pl.ds(s * RPS, RPS)],
                      nd_h.at[c, g, pl.ds(s * RPS, RPS)])

  return body(idx2, ce, stab, dtab, zrow)


def _padcol(a, width):
  return jnp.pad(a, ((0, 0), (0, width - a.shape[1])))


def _padvec(v, width):
  return jnp.pad(v, (0, width - v.shape[0]))


def _run(edge_index, nodes_feat, edges_feat, snorm_n, snorm_e, params):
  f32 = jnp.float32
  src = edge_index[0]
  dst = edge_index[1]

  whp = _padcol(params["Wh"], DP)
  bhp = _padvec(params["bh"], DP)[None, :]
  lp = params["layers"]

  def pack_layer(p):
    wp = jnp.concatenate(
        [_padcol(p[k + "_W"], DP) for k in "ABDE"], axis=1)
    wp = jnp.pad(wp, ((0, DP - D), (0, 0)))
    bp = jnp.concatenate([_padvec(p[k + "_b"], DP) for k in "ABDE"])[None, :]
    return wp, bp

  wp1, bp1 = pack_layer(lp[0])
  wp2, bp2 = pack_layer(lp[1])
  w1fold = _padvec(params["be"] @ lp[0]["C_W"] + lp[0]["C_b"], DP)
  bp1 = bp1.at[0, 3 * DP:4 * DP].add(w1fold)

  we_row = _padvec(params["We"][0], DP)[None, :]
  bep = _padvec(params["be"], DP)[None, :]
  u1 = _padvec(params["We"][0] @ lp[0]["C_W"], DP)
  c2w = jnp.pad(_padcol(lp[1]["C_W"], DP), ((0, DP - D), (0, 0)))
  c2b = _padvec(lp[1]["C_b"], DP)[None, :]
  bnh1_g = _padvec(lp[0]["bnh_g"], DP)[None, :]
  bnh1_b = _padvec(lp[0]["bnh_b"], DP)[None, :]
  bne1_g = _padvec(lp[0]["bne_g"], DP)[None, :]
  bne1_b = _padvec(lp[0]["bne_b"], DP)[None, :]
  bnh2_g = _padvec(lp[1]["bnh_g"], DP)[None, :]
  bnh2_b = _padvec(lp[1]["bnh_b"], DP)[None, :]

  npad = EPAD - E
  srcp = jnp.concatenate([src, jnp.zeros((npad,), jnp.int32)]).reshape(
      NW, NBLK, BK)
  dstp = jnp.concatenate([dst, jnp.full((npad,), N, jnp.int32)]).reshape(
      NW, NBLK, BK)
  efp = jnp.concatenate([edges_feat[:, 0],
                         jnp.zeros((npad,), f32)]).reshape(NW, NBLK, BK)
  snp = jnp.concatenate([snorm_e[:, 0],
                         jnp.zeros((npad,), f32)]).reshape(NW, NBLK, BK)
  idx4 = jnp.stack([
      srcp, dstp,
      jax.lax.bitcast_convert_type(efp, jnp.int32),
      jax.lax.bitcast_convert_type(snp, jnp.int32)], axis=2)
  idx2 = jnp.stack([srcp, dstp], axis=2)
  ef2d = jnp.concatenate([edges_feat[:, 0],
                          jnp.zeros((npad,), f32)])[:, None]
  zrow = jnp.zeros((RPS, 32), f32)

  h0, a1, s1, d1 = _entry_call(nodes_feat, whp, bhp, wp1, bp1)
  stab1 = s1.reshape(G * N, 32)
  dtab1 = jnp.pad(d1, ((0, 0), (0, NT - N), (0, 0))).reshape(G * NT, 16)
  nd1, t1, st1 = _sc_layer1(idx4, stab1, dtab1, u1, zrow)
  hnew1, hst1, ste1 = _hnew_call(a1, nd1, snorm_n, st1)
  h1, a2, s2, d2 = _hfin_call(h0, hnew1, hst1, bnh1_g, bnh1_b, wp2, bp2)

  ce2 = _ce2_call(t1, ef2d, ste1, we_row, bep, bne1_g, bne1_b, c2w, c2b)
  stab2 = s2.reshape(G * N, 32)
  dtab2 = jnp.pad(d2, ((0, 0), (0, NT - N), (0, 0))).reshape(G * NT, 16)
  nd2 = _sc_layer2(idx2, ce2, stab2, dtab2, zrow)
  hnew2, hst2, _ = _hnew_call(a2, nd2, snorm_n, st1)
  out = _final_call(h1, hnew2, hst2, bnh2_g, bnh2_b)
  return out[:, :D]


_run_jit = jax.jit(_run)


def kernel(edge_index, nodes_feat, edges_feat, nodes_num_norm_sqrt,
           edges_num_norm_sqrt, params):
  return _run_jit(edge_index, nodes_feat, edges_feat, nodes_num_norm_sqrt,
              edges_num_norm_sqrt, params)


# speedup vs baseline: 4.0337x; 1.0338x over previous
import functools

import jax
import jax.numpy as jnp
from jax import lax
from jax.experimental import pallas as pl
from jax.experimental.pallas import tpu as pltpu
from jax.experimental.pallas import tpu_sc as plsc

N = 50000
E = 800000
IN_DIM = 64
D = 70
DP = 80
G = 5
NW = 32
BK = 128
NBLK = 196
EPT = NBLK * BK
EPAD = NW * EPT
NT = 50016
NTT = 52000
RPS = NT // 16


BN_NODE = 2000
NSTEPS = N // BN_NODE


def _pack_tables(tabs):
  Bh = tabs[:, DP:2 * DP]
  Dh = tabs[:, 2 * DP:3 * DP]
  Eh = tabs[:, 3 * DP:4 * DP]
  src = jnp.stack([
      jnp.concatenate([Bh[:, 16 * g:16 * (g + 1)], Dh[:, 16 * g:16 * (g + 1)]],
                      axis=1) for g in range(G)], axis=0)
  dst = jnp.stack([Eh[:, 16 * g:16 * (g + 1)] for g in range(G)], axis=0)
  return src, dst


def _entry_body(x_ref, wh_ref, bh_ref, wp_ref, bp_ref,
                h0_ref, a_ref, s_ref, d_ref):
  h0 = jnp.dot(x_ref[...], wh_ref[...],
               preferred_element_type=jnp.float32) + bh_ref[...]
  tabs = jnp.dot(h0, wp_ref[...],
                 preferred_element_type=jnp.float32) + bp_ref[...]
  h0_ref[...] = h0
  a_ref[...] = tabs[:, :DP]
  s, d = _pack_tables(tabs)
  s_ref[...] = s
  d_ref[...] = jnp.where(pl.program_id(0) < NSTEPS, d, 0.0)


def _entry_call(x, wh, bh, wp, bp):
  return pl.pallas_call(
      _entry_body,
      grid=(NSTEPS + 1,),
      in_specs=[
          pl.BlockSpec((BN_NODE, IN_DIM),
                       lambda i: (jnp.minimum(i, NSTEPS - 1), 0)),
          pl.BlockSpec((IN_DIM, DP), lambda i: (0, 0)),
          pl.BlockSpec((1, DP), lambda i: (0, 0)),
          pl.BlockSpec((DP, 4 * DP), lambda i: (0, 0)),
          pl.BlockSpec((1, 4 * DP), lambda i: (0, 0)),
      ],
      out_specs=[
          pl.BlockSpec((BN_NODE, DP),
                       lambda i: (jnp.minimum(i, NSTEPS - 1), 0)),
          pl.BlockSpec((BN_NODE, DP),
                       lambda i: (jnp.minimum(i, NSTEPS - 1), 0)),
          pl.BlockSpec((G, BN_NODE, 32),
                       lambda i: (0, jnp.minimum(i, NSTEPS - 1), 0)),
          pl.BlockSpec((G, BN_NODE, 16), lambda i: (0, i, 0)),
      ],
      out_shape=[
          jax.ShapeDtypeStruct((N, DP), jnp.float32),
          jax.ShapeDtypeStruct((N, DP), jnp.float32),
          jax.ShapeDtypeStruct((G, N, 32), jnp.float32),
          jax.ShapeDtypeStruct((G, NTT, 16), jnp.float32),
      ],
  )(x, wh, bh, wp, bp)


def _hnew_body(a_ref, nd_ref, snn_ref, est_ref, h_ref, st_ref, ste_ref,
               acc_ref):
  i = pl.program_id(0)
  num = jnp.concatenate(
      [nd_ref[0, g, :, 0:16] + nd_ref[1, g, :, 0:16] for g in range(G)],
      axis=1)
  den = jnp.concatenate(
      [nd_ref[0, g, :, 16:32] + nd_ref[1, g, :, 16:32] for g in range(G)],
      axis=1)
  hnew = (a_ref[...] + num / (den + 1e-6)) * snn_ref[...]
  h_ref[...] = hnew

  @pl.when(i == 0)
  def _():
    acc_ref[...] = jnp.zeros_like(acc_ref)

  acc_ref[0:1, :] += jnp.sum(hnew, axis=0, keepdims=True)
  acc_ref[1:2, :] += jnp.sum(hnew * hnew, axis=0, keepdims=True)

  @pl.when(i == NSTEPS - 1)
  def _():
    st_ref[...] = acc_ref[...]
    sums = jnp.concatenate(
        [jnp.sum(est_ref[:, :, g, 0:16], axis=(0, 1)) for g in range(G)])
    sqs = jnp.concatenate(
        [jnp.sum(est_ref[:, :, g, 16:32], axis=(0, 1)) for g in range(G)])
    ste_ref[...] = jnp.stack([sums, sqs], axis=0)


def _hnew_call(a, nd, snn, est):
  return pl.pallas_call(
      _hnew_body,
      grid=(NSTEPS,),
      in_specs=[
          pl.BlockSpec((BN_NODE, DP), lambda i: (i, 0)),
          pl.BlockSpec((2, G, BN_NODE, 32), lambda i: (0, 0, i, 0)),
          pl.BlockSpec((BN_NODE, 1), lambda i: (i, 0)),
          pl.BlockSpec((2, 16, G, 32), lambda i: (0, 0, 0, 0)),
      ],
      out_specs=[
          pl.BlockSpec((BN_NODE, DP), lambda i: (i, 0)),
          pl.BlockSpec((2, DP), lambda i: (0, 0)),
          pl.BlockSpec((2, DP), lambda i: (0, 0)),
      ],
      out_shape=[
          jax.ShapeDtypeStruct((N, DP), jnp.float32),
          jax.ShapeDtypeStruct((2, DP), jnp.float32),
          jax.ShapeDtypeStruct((2, DP), jnp.float32),
      ],
      scratch_shapes=[pltpu.VMEM((2, DP), jnp.float32)],
  )(a, nd, snn, est)


def _hfin_body(hprev_ref, hnew_ref, st_ref, g_ref, b_ref, wp_ref, bp_ref,
               h_ref, a_ref, s_ref, d_ref):
  m = st_ref[0:1, :] / N
  v = st_ref[1:2, :] / N - m * m
  hn = g_ref[...] * (hnew_ref[...] - m) * lax.rsqrt(v + 1e-5) + b_ref[...]
  h1 = hprev_ref[...] + jnp.maximum(hn, 0.0)
  h_ref[...] = h1
  tabs = jnp.dot(h1, wp_ref[...],
                 preferred_element_type=jnp.float32) + bp_ref[...]
  a_ref[...] = tabs[:, :DP]
  s, d = _pack_tables(tabs)
  s_ref[...] = s
  d_ref[...] = jnp.where(pl.program_id(0) < NSTEPS, d, 0.0)


def _hfin_call(hprev, hnew, st, g, b, wp, bp):
  return pl.pallas_call(
      _hfin_body,
      grid=(NSTEPS + 1,),
      in_specs=[
          pl.BlockSpec((BN_NODE, DP),
                       lambda i: (jnp.minimum(i, NSTEPS - 1), 0)),
          pl.BlockSpec((BN_NODE, DP),
                       lambda i: (jnp.minimum(i, NSTEPS - 1), 0)),
          pl.BlockSpec((2, DP), lambda i: (0, 0)),
          pl.BlockSpec((1, DP), lambda i: (0, 0)),
          pl.BlockSpec((1, DP), lambda i: (0, 0)),
          pl.BlockSpec((DP, 4 * DP), lambda i: (0, 0)),
          pl.BlockSpec((1, 4 * DP), lambda i: (0, 0)),
      ],
      out_specs=[
          pl.BlockSpec((BN_NODE, DP),
                       lambda i: (jnp.minimum(i, NSTEPS - 1), 0)),
          pl.BlockSpec((BN_NODE, DP),
                       lambda i: (jnp.minimum(i, NSTEPS - 1), 0)),
          pl.BlockSpec((G, BN_NODE, 32),
                       lambda i: (0, jnp.minimum(i, NSTEPS - 1), 0)),
          pl.BlockSpec((G, BN_NODE, 16), lambda i: (0, i, 0)),
      ],
      out_shape=[
          jax.ShapeDtypeStruct((N, DP), jnp.float32),
          jax.ShapeDtypeStruct((N, DP), jnp.float32),
          jax.ShapeDtypeStruct((G, N, 32), jnp.float32),
          jax.ShapeDtypeStruct((G, NTT, 16), jnp.float32),
      ],
  )(hprev, hnew, st, g, b, wp, bp)


BE = 2048
ESTEPS = EPAD // BE


def _ce2_body(t_ref, ef_ref, ste_ref, werow_ref, be_ref, g1_ref, b1_ref,
              cw_ref, cb_ref, out_ref):
  m = ste_ref[0:1, :] / E
  v = ste_ref[1:2, :] / E - m * m
  bn = g1_ref[...] * (t_ref[...] - m) * lax.rsqrt(v + 1e-5) + b1_ref[...]
  e1 = ef_ref[...] * werow_ref[...] + be_ref[...] + jnp.maximum(bn, 0.0)
  out_ref[...] = jnp.dot(e1, cw_ref[...],
                         preferred_element_type=jnp.float32) + cb_ref[...]


def _ce2_call(t, ef, ste, werow, be, g1, b1, cw, cb):
  return pl.pallas_call(
      _ce2_body,
      grid=(ESTEPS,),
      in_specs=[
          pl.BlockSpec((BE, DP), lambda i: (i, 0)),
          pl.BlockSpec((BE, 1), lambda i: (i, 0)),
          pl.BlockSpec((2, DP), lambda i: (0, 0)),
          pl.BlockSpec((1, DP), lambda i: (0, 0)),
          pl.BlockSpec((1, DP), lambda i: (0, 0)),
          pl.BlockSpec((1, DP), lambda i: (0, 0)),
          pl.BlockSpec((1, DP), lambda i: (0, 0)),
          pl.BlockSpec((DP, DP), lambda i: (0, 0)),
          pl.BlockSpec((1, DP), lambda i: (0, 0)),
      ],
      out_specs=pl.BlockSpec((BE, DP), lambda i: (i, 0)),
      out_shape=jax.ShapeDtypeStruct((EPAD, DP), jnp.float32),
  )(t, ef, ste, werow, be, g1, b1, cw, cb)


def _final_body(hprev_ref, hnew_ref, st_ref, g_ref, b_ref, out_ref, acc_ref):
  i = pl.program_id(0)
  m = st_ref[0:1, :] / N
  v = st_ref[1:2, :] / N - m * m
  hn = g_ref[...] * (hnew_ref[...] - m) * lax.rsqrt(v + 1e-5) + b_ref[...]
  h2 = hprev_ref[...] + jnp.maximum(hn, 0.0)

  @pl.when(i == 0)
  def _():
    acc_ref[...] = jnp.zeros_like(acc_ref)

  acc_ref[...] += jnp.sum(h2, axis=0, keepdims=True)

  @pl.when(i == NSTEPS - 1)
  def _():
    out_ref[...] = acc_ref[...] / N


def _final_call(hprev, hnew, st, g, b):
  return pl.pallas_call(
      _final_body,
      grid=(NSTEPS,),
      in_specs=[
          pl.BlockSpec((BN_NODE, DP), lambda i: (i, 0)),
          pl.BlockSpec((BN_NODE, DP), lambda i: (i, 0)),
          pl.BlockSpec((2, DP), lambda i: (0, 0)),
          pl.BlockSpec((1, DP), lambda i: (0, 0)),
          pl.BlockSpec((1, DP), lambda i: (0, 0)),
      ],
      out_specs=pl.BlockSpec((1, DP), lambda i: (0, 0)),
      out_shape=jax.ShapeDtypeStruct((1, DP), jnp.float32),
      scratch_shapes=[pltpu.VMEM((1, DP), jnp.float32)],
  )(hprev, hnew, st, g, b)


@functools.cache
def _sc_mesh():
  return plsc.VectorSubcoreMesh(core_axis_name="c", subcore_axis_name="s",
                                num_cores=2, num_subcores=16)


def _sc_layer1(srcp, dstp, efp, snp, stab, dtab, u, zrow):
  out_type = (
      jax.ShapeDtypeStruct((2, G, NT, 32), jnp.float32),
      jax.ShapeDtypeStruct((EPAD, DP), jnp.float32),
      jax.ShapeDtypeStruct((2, 16, G, 32), jnp.float32),
  )
  scratch = [
      pltpu.VMEM_SHARED((NT, 32), jnp.float32),
      [pltpu.VMEM((BK,), jnp.int32) for _ in range(2)],
      [pltpu.VMEM((BK,), jnp.int32) for _ in range(2)],
      [pltpu.VMEM((BK,), jnp.float32) for _ in range(2)],
      [pltpu.VMEM((BK,), jnp.float32) for _ in range(2)],
      [pltpu.VMEM((BK,), jnp.int32) for _ in range(2)],
      [pltpu.VMEM((BK,), jnp.int32) for _ in range(2)],
      [pltpu.VMEM((BK,), jnp.int32) for _ in range(2)],
      [pltpu.VMEM((BK,), jnp.float32) for _ in range(2)],
      [pltpu.VMEM((BK,), jnp.float32) for _ in range(2)],
      [pltpu.VMEM((BK, 32), jnp.float32) for _ in range(2)],
      [pltpu.VMEM((BK, 16), jnp.float32) for _ in range(2)],
      [pltpu.VMEM((BK, 32), jnp.float32) for _ in range(2)],
      [pltpu.VMEM((BK, 16), jnp.float32) for _ in range(2)],
      pltpu.VMEM((DP,), jnp.float32),
      pltpu.VMEM((32,), jnp.float32),
      [pltpu.SemaphoreType.DMA for _ in range(2)],
      [pltpu.SemaphoreType.DMA for _ in range(2)],
      [pltpu.SemaphoreType.DMA for _ in range(2)],
      [pltpu.SemaphoreType.DMA for _ in range(2)],
  ]

  @functools.partial(pl.kernel, out_type=out_type, mesh=_sc_mesh(),
                     scratch_types=scratch,
                     compiler_params=pltpu.CompilerParams(
                         use_tc_tiling_on_sc=False,
                         needs_layout_passes=False))
  def body(srcp_h, dstp_h, efp_h, snp_h, stab_h, dtab_h, u_h, z_h,
           nd_h, t_h, st_h,
           acc, SIDXL, DIDXL, EFL, SNL, SGI, GDI, DSTS, EFS, SNS,
           SV, DV, PV, TV, uv, stbuf, sl, sg, ss, st):
    c = lax.axis_index("c")
    s = lax.axis_index("s")
    wid = c * 16 + s
    ebase = wid * EPT
    pltpu.sync_copy(u_h, uv)
    z16 = jnp.zeros((16,), jnp.float32)

    for g in range(G):
      pltpu.sync_copy(z_h, acc.at[pl.ds(s * RPS, RPS)])
      plsc.subcore_barrier()
      ug = uv[pl.ds(16 * g, 16)]

      def t_dst(bb):
        return t_h.at[pl.ds(ebase + bb * BK, BK), pl.ds(16 * g, 16)]

      def body2(j, carry):
        for par in range(2):
          bb = 2 * j + par
          p_i = (par + 1) % 2
          p_c = par

          @pl.when(bb < NBLK)
          def _():
            pltpu.async_copy(srcp_h.at[wid, bb], SIDXL[p_c], sl[p_c])
            pltpu.async_copy(dstp_h.at[wid, bb], DIDXL[p_c], sl[p_c])
            pltpu.async_copy(efp_h.at[wid, bb], EFL[p_c], sl[p_c])
            pltpu.async_copy(snp_h.at[wid, bb], SNL[p_c], sl[p_c])

          @pl.when(jnp.logical_and(bb >= 3, bb < NBLK + 3))
          def _():
            pltpu.make_async_copy(PV[p_i], acc.at[DSTS[p_i]], ss[p_i]).wait()
            pltpu.make_async_copy(TV[p_i], t_dst(bb - 3), st[p_i]).wait()

          @pl.when(jnp.logical_and(bb >= 1, bb < NBLK + 1))
          def _():
            pltpu.make_async_copy(
                srcp_h.at[wid, bb - 1], SIDXL[p_i], sl[p_i]).wait()
            pltpu.make_async_copy(
                dstp_h.at[wid, bb - 1], DIDXL[p_i], sl[p_i]).wait()
            pltpu.make_async_copy(
                efp_h.at[wid, bb - 1], EFL[p_i], sl[p_i]).wait()
            pltpu.make_async_copy(
                snp_h.at[wid, bb - 1], SNL[p_i], sl[p_i]).wait()
            for kk in range(BK // 16):
              sl16 = pl.ds(kk * 16, 16)
              srow = SIDXL[p_i][sl16]
              drow = DIDXL[p_i][sl16]
              SGI[p_i][sl16] = srow + g * N
              GDI[p_i][sl16] = drow + g * NTT
              DSTS[p_i][sl16] = drow
              EFS[p_i][sl16] = EFL[p_i][sl16]
              SNS[p_i][sl16] = SNL[p_i][sl16]
            pltpu.async_copy(stab_h.at[SGI[p_i]], SV[p_i], sg[p_i])
            pltpu.async_copy(dtab_h.at[GDI[p_i]], DV[p_i], sg[p_i])

          def c_stage(ec):
            pltpu.make_async_copy(stab_h.at[SGI[p_c]], SV[p_c], sg[p_c]).wait()
            pltpu.make_async_copy(dtab_h.at[GDI[p_c]], DV[p_c], sg[p_c]).wait()

            @plsc.parallel_loop(0, BK, unroll=4, carry=ec)
            def edge_loop(i, ec2):
              ssum, ssq = ec2
              bh = SV[p_c][i, pl.ds(0, 16)]
              dh = SV[p_c][i, pl.ds(16, 16)]
              eh = DV[p_c][i, pl.ds(0, 16)]
              ifull = jnp.full((16,), i, jnp.int32)
              efb = plsc.load_gather(EFS[p_c], [ifull])
              snb = plsc.load_gather(SNS[p_c], [ifull])
              en = efb * ug + dh + eh
              sig = 1.0 / (1.0 + jnp.exp(-en))
              PV[p_c][i, pl.ds(0, 16)] = sig * bh
              PV[p_c][i, pl.ds(16, 16)] = sig
              t = en * snb
              TV[p_c][i, pl.ds(0, 16)] = t
              return ssum + t, ssq + t * t

            ec = edge_loop
            pltpu.async_copy(PV[p_c], acc.at[DSTS[p_c]], ss[p_c], add=True)
            pltpu.async_copy(TV[p_c], t_dst(bb - 2), st[p_c])
            return ec

          carry = lax.cond(
              jnp.logical_and(bb >= 2, bb < NBLK + 2),
              c_stage, lambda ec: ec, carry)
        return carry

      ssum, ssq = lax.fori_loop(0, (NBLK + 4) // 2, body2, (z16, z16))
      stbuf[pl.ds(0, 16)] = ssum
      stbuf[pl.ds(16, 16)] = ssq
      pltpu.sync_copy(stbuf, st_h.at[c, s, g])
      plsc.subcore_barrier()
      pltpu.sync_copy(acc.at[pl.ds(s * RPS, RPS)],
                      nd_h.at[c, g, pl.ds(s * RPS, RPS)])

  return body(srcp, dstp, efp, snp, stab, dtab, u, zrow)


def _sc_layer2(srcp, dstp, ce, stab, dtab, zrow):
  out_type = jax.ShapeDtypeStruct((2, G, NT, 32), jnp.float32)
  scratch = [
      pltpu.VMEM_SHARED((NT, 32), jnp.float32),
      [pltpu.VMEM((BK,), jnp.int32) for _ in range(2)],
      [pltpu.VMEM((BK,), jnp.int32) for _ in range(2)],
      [pltpu.VMEM((BK,), jnp.int32) for _ in range(2)],
      [pltpu.VMEM((BK,), jnp.int32) for _ in range(2)],
      [pltpu.VMEM((BK,), jnp.int32) for _ in range(2)],
      [pltpu.VMEM((BK, 32), jnp.float32) for _ in range(2)],
      [pltpu.VMEM((BK, 16), jnp.float32) for _ in range(2)],
      [pltpu.VMEM((BK, 16), jnp.float32) for _ in range(2)],
      [pltpu.VMEM((BK, 32), jnp.float32) for _ in range(2)],
      [pltpu.SemaphoreType.DMA for _ in range(2)],
      [pltpu.SemaphoreType.DMA for _ in range(2)],
      [pltpu.SemaphoreType.DMA for _ in range(2)],
  ]

  @functools.partial(pl.kernel, out_type=out_type, mesh=_sc_mesh(),
                     scratch_types=scratch,
                     compiler_params=pltpu.CompilerParams(
                         use_tc_tiling_on_sc=False,
                         needs_layout_passes=False))
  def body(srcp_h, dstp_h, ce_h, stab_h, dtab_h, z_h, nd_h,
           acc, SIDXL, DIDXL, SGI, GDI, DSTS, SV, DV, CV, PV, sl, sg, ss):
    c = lax.axis_index("c")
    s = lax.axis_index("s")
    wid = c * 16 + s
    ebase = wid * EPT

    for g in range(G):
      pltpu.sync_copy(z_h, acc.at[pl.ds(s * RPS, RPS)])
      plsc.subcore_barrier()

      def ce_src(bb):
        return ce_h.at[pl.ds(ebase + bb * BK, BK), pl.ds(16 * g, 16)]

      def body2(j, carry):
        for par in range(2):
          bb = 2 * j + par
          p_i = (par + 1) % 2
          p_c = par

          @pl.when(bb < NBLK)
          def _():
            pltpu.async_copy(srcp_h.at[wid, bb], SIDXL[p_c], sl[p_c])
            pltpu.async_copy(dstp_h.at[wid, bb], DIDXL[p_c], sl[p_c])

          @pl.when(jnp.logical_and(bb >= 3, bb < NBLK + 3))
          def _():
            pltpu.make_async_copy(PV[p_i], acc.at[DSTS[p_i]], ss[p_i]).wait()

          @pl.when(jnp.logical_and(bb >= 1, bb < NBLK + 1))
          def _():
            pltpu.make_async_copy(
                srcp_h.at[wid, bb - 1], SIDXL[p_i], sl[p_i]).wait()
            pltpu.make_async_copy(
                dstp_h.at[wid, bb - 1], DIDXL[p_i], sl[p_i]).wait()
            for kk in range(BK // 16):
              sl16 = pl.ds(kk * 16, 16)
              srow = SIDXL[p_i][sl16]
              drow = DIDXL[p_i][sl16]
              SGI[p_i][sl16] = srow + g * N
              GDI[p_i][sl16] = drow + g * NTT
              DSTS[p_i][sl16] = drow
            pltpu.async_copy(stab_h.at[SGI[p_i]], SV[p_i], sg[p_i])
            pltpu.async_copy(dtab_h.at[GDI[p_i]], DV[p_i], sg[p_i])
            pltpu.async_copy(ce_src(bb - 1), CV[p_i], sg[p_i])

          @pl.when(jnp.logical_and(bb >= 2, bb < NBLK + 2))
          def _():
            pltpu.make_async_copy(stab_h.at[SGI[p_c]], SV[p_c], sg[p_c]).wait()
            pltpu.make_async_copy(dtab_h.at[GDI[p_c]], DV[p_c], sg[p_c]).wait()
            pltpu.make_async_copy(ce_src(bb - 2), CV[p_c], sg[p_c]).wait()

            @plsc.parallel_loop(0, BK, unroll=4)
            def edge_loop(i):
              bh = SV[p_c][i, pl.ds(0, 16)]
              dh = SV[p_c][i, pl.ds(16, 16)]
              eh = DV[p_c][i, pl.ds(0, 16)]
              en = CV[p_c][i, pl.ds(0, 16)] + dh + eh
              sig = 1.0 / (1.0 + jnp.exp(-en))
              PV[p_c][i, pl.ds(0, 16)] = sig * bh
              PV[p_c][i, pl.ds(16, 16)] = sig

            pltpu.async_copy(PV[p_c], acc.at[DSTS[p_c]], ss[p_c], add=True)
        return carry

      lax.fori_loop(0, (NBLK + 4) // 2, body2, 0)
      plsc.subcore_barrier()
      pltpu.sync_copy(acc.at[pl.ds(s * RPS, RPS)],
                      nd_h.at[c, g, pl.ds(s * RPS, RPS)])

  return body(srcp, dstp, ce, stab, dtab, zrow)


def _padcol(a, width):
  return jnp.pad(a, ((0, 0), (0, width - a.shape[1])))


def _padvec(v, width):
  return jnp.pad(v, (0, width - v.shape[0]))


def _run(edge_index, nodes_feat, edges_feat, snorm_n, snorm_e, params):
  f32 = jnp.float32
  src = edge_index[0]
  dst = edge_index[1]

  whp = _padcol(params["Wh"], DP)
  bhp = _padvec(params["bh"], DP)[None, :]
  lp = params["layers"]

  def pack_layer(p):
    wp = jnp.concatenate(
        [_padcol(p[k + "_W"], DP) for k in "ABDE"], axis=1)
    wp = jnp.pad(wp, ((0, DP - D), (0, 0)))
    bp = jnp.concatenate([_padvec(p[k + "_b"], DP) for k in "ABDE"])[None, :]
    return wp, bp

  wp1, bp1 = pack_layer(lp[0])
  wp2, bp2 = pack_layer(lp[1])
  w1fold = _padvec(params["be"] @ lp[0]["C_W"] + lp[0]["C_b"], DP)
  bp1 = bp1.at[0, 3 * DP:4 * DP].add(w1fold)

  we_row = _padvec(params["We"][0], DP)[None, :]
  bep = _padvec(params["be"], DP)[None, :]
  u1 = _padvec(params["We"][0] @ lp[0]["C_W"], DP)
  c2w = jnp.pad(_padcol(lp[1]["C_W"], DP), ((0, DP - D), (0, 0)))
  c2b = _padvec(lp[1]["C_b"], DP)[None, :]
  bnh1_g = _padvec(lp[0]["bnh_g"], DP)[None, :]
  bnh1_b = _padvec(lp[0]["bnh_b"], DP)[None, :]
  bne1_g = _padvec(lp[0]["bne_g"], DP)[None, :]
  bne1_b = _padvec(lp[0]["bne_b"], DP)[None, :]
  bnh2_g = _padvec(lp[1]["bnh_g"], DP)[None, :]
  bnh2_b = _padvec(lp[1]["bnh_b"], DP)[None, :]

  npad = EPAD - E
  srcp = jnp.concatenate([src, jnp.zeros((npad,), jnp.int32)]).reshape(
      NW, NBLK, BK)
  dstp = jnp.concatenate([dst, jnp.full((npad,), N, jnp.int32)]).reshape(
      NW, NBLK, BK)
  efp = jnp.concatenate([edges_feat[:, 0],
                         jnp.zeros((npad,), f32)]).reshape(NW, NBLK, BK)
  snp = jnp.concatenate([snorm_e[:, 0],
                         jnp.zeros((npad,), f32)]).reshape(NW, NBLK, BK)
  ef2d = efp.reshape(EPAD, 1)
  zrow = jnp.zeros((RPS, 32), f32)

  h0, a1, s1, d1 = _entry_call(nodes_feat, whp, bhp, wp1, bp1)
  stab1 = s1.reshape(G * N, 32)
  dtab1 = d1.reshape(G * NTT, 16)
  nd1, t1, st1 = _sc_layer1(srcp, dstp, efp, snp, stab1, dtab1, u1, zrow)
  hnew1, hst1, ste1 = _hnew_call(a1, nd1, snorm_n, st1)
  h1, a2, s2, d2 = _hfin_call(h0, hnew1, hst1, bnh1_g, bnh1_b, wp2, bp2)

  ce2 = _ce2_call(t1, ef2d, ste1, we_row, bep, bne1_g, bne1_b, c2w, c2b)
  stab2 = s2.reshape(G * N, 32)
  dtab2 = d2.reshape(G * NTT, 16)
  nd2 = _sc_layer2(srcp, dstp, ce2, stab2, dtab2, zrow)
  hnew2, hst2, _ = _hnew_call(a2, nd2, snorm_n, st1)
  out = _final_call(h1, hnew2, hst2, bnh2_g, bnh2_b)
  return out[:, :D]


_run_jit = jax.jit(_run)


def kernel(edge_index, nodes_feat, edges_feat, nodes_num_norm_sqrt,
           edges_num_norm_sqrt, params):
  return _run_jit(edge_index, nodes_feat, edges_feat, nodes_num_norm_sqrt,
              edges_num_norm_sqrt, params)


# speedup vs baseline: 4.0344x; 1.0002x over previous
import functools

import jax
import jax.numpy as jnp
from jax import lax
from jax.experimental import pallas as pl
from jax.experimental.pallas import tpu as pltpu
from jax.experimental.pallas import tpu_sc as plsc

N = 50000
E = 800000
IN_DIM = 64
D = 70
DP = 80
G = 5
NW = 32
BK = 128
NBLK = 196
EPT = NBLK * BK
EPAD = NW * EPT
NT = 50016
NTT = 52000
RPS = NT // 16


BN_NODE = 2000
NSTEPS = N // BN_NODE


def _pack_tables(tabs):
  Bh = tabs[:, DP:2 * DP]
  Dh = tabs[:, 2 * DP:3 * DP]
  Eh = tabs[:, 3 * DP:4 * DP]
  src = jnp.stack([
      jnp.concatenate([Bh[:, 16 * g:16 * (g + 1)], Dh[:, 16 * g:16 * (g + 1)]],
                      axis=1) for g in range(G)], axis=0)
  dst = jnp.stack([Eh[:, 16 * g:16 * (g + 1)] for g in range(G)], axis=0)
  return src, dst


def _entry_body(x_ref, wh_ref, bh_ref, wp_ref, bp_ref,
                h0_ref, a_ref, s_ref, d_ref):
  h0 = jnp.dot(x_ref[...], wh_ref[...],
               preferred_element_type=jnp.float32) + bh_ref[...]
  tabs = jnp.dot(h0, wp_ref[...],
                 preferred_element_type=jnp.float32) + bp_ref[...]
  h0_ref[...] = h0
  a_ref[...] = tabs[:, :DP]
  s, d = _pack_tables(tabs)
  s_ref[...] = s
  d_ref[...] = jnp.where(pl.program_id(0) < NSTEPS, d, 0.0)


def _entry_call(x, wh, bh, wp, bp):
  return pl.pallas_call(
      _entry_body,
      grid=(NSTEPS + 1,),
      in_specs=[
          pl.BlockSpec((BN_NODE, IN_DIM),
                       lambda i: (jnp.minimum(i, NSTEPS - 1), 0)),
          pl.BlockSpec((IN_DIM, DP), lambda i: (0, 0)),
          pl.BlockSpec((1, DP), lambda i: (0, 0)),
          pl.BlockSpec((DP, 4 * DP), lambda i: (0, 0)),
          pl.BlockSpec((1, 4 * DP), lambda i: (0, 0)),
      ],
      out_specs=[
          pl.BlockSpec((BN_NODE, DP),
                       lambda i: (jnp.minimum(i, NSTEPS - 1), 0)),
          pl.BlockSpec((BN_NODE, DP),
                       lambda i: (jnp.minimum(i, NSTEPS - 1), 0)),
          pl.BlockSpec((G, BN_NODE, 32),
                       lambda i: (0, jnp.minimum(i, NSTEPS - 1), 0)),
          pl.BlockSpec((G, BN_NODE, 16), lambda i: (0, i, 0)),
      ],
      out_shape=[
          jax.ShapeDtypeStruct((N, DP), jnp.float32),
          jax.ShapeDtypeStruct((N, DP), jnp.float32),
          jax.ShapeDtypeStruct((G, N, 32), jnp.float32),
          jax.ShapeDtypeStruct((G, NTT, 16), jnp.float32),
      ],
  )(x, wh, bh, wp, bp)


def _hnew_body(a_ref, nd_ref, snn_ref, est_ref, h_ref, st_ref, ste_ref,
               acc_ref):
  i = pl.program_id(0)
  num = jnp.concatenate(
      [nd_ref[0, g, :, 0:16] + nd_ref[1, g, :, 0:16] for g in range(G)],
      axis=1)
  den = jnp.concatenate(
      [nd_ref[0, g, :, 16:32] + nd_ref[1, g, :, 16:32] for g in range(G)],
      axis=1)
  hnew = (a_ref[...] + num / (den + 1e-6)) * snn_ref[...]
  h_ref[...] = hnew

  @pl.when(i == 0)
  def _():
    acc_ref[...] = jnp.zeros_like(acc_ref)

  acc_ref[0:1, :] += jnp.sum(hnew, axis=0, keepdims=True)
  acc_ref[1:2, :] += jnp.sum(hnew * hnew, axis=0, keepdims=True)

  @pl.when(i == NSTEPS - 1)
  def _():
    st_ref[...] = acc_ref[...]
    sums = jnp.concatenate(
        [jnp.sum(est_ref[:, :, g, 0:16], axis=(0, 1)) for g in range(G)])
    sqs = jnp.concatenate(
        [jnp.sum(est_ref[:, :, g, 16:32], axis=(0, 1)) for g in range(G)])
    ste_ref[...] = jnp.stack([sums, sqs], axis=0)


def _hnew_call(a, nd, snn, est):
  return pl.pallas_call(
      _hnew_body,
      grid=(NSTEPS,),
      in_specs=[
          pl.BlockSpec((BN_NODE, DP), lambda i: (i, 0)),
          pl.BlockSpec((2, G, BN_NODE, 32), lambda i: (0, 0, i, 0)),
          pl.BlockSpec((BN_NODE, 1), lambda i: (i, 0)),
          pl.BlockSpec((2, 16, G, 32), lambda i: (0, 0, 0, 0)),
      ],
      out_specs=[
          pl.BlockSpec((BN_NODE, DP), lambda i: (i, 0)),
          pl.BlockSpec((2, DP), lambda i: (0, 0)),
          pl.BlockSpec((2, DP), lambda i: (0, 0)),
      ],
      out_shape=[
          jax.ShapeDtypeStruct((N, DP), jnp.float32),
          jax.ShapeDtypeStruct((2, DP), jnp.float32),
          jax.ShapeDtypeStruct((2, DP), jnp.float32),
      ],
      scratch_shapes=[pltpu.VMEM((2, DP), jnp.float32)],
  )(a, nd, snn, est)


def _hfin_body(hprev_ref, hnew_ref, st_ref, g_ref, b_ref, wp_ref, bp_ref,
               h_ref, a_ref, s_ref, d_ref):
  m = st_ref[0:1, :] / N
  v = st_ref[1:2, :] / N - m * m
  hn = g_ref[...] * (hnew_ref[...] - m) * lax.rsqrt(v + 1e-5) + b_ref[...]
  h1 = hprev_ref[...] + jnp.maximum(hn, 0.0)
  h_ref[...] = h1
  tabs = jnp.dot(h1, wp_ref[...],
                 preferred_element_type=jnp.float32) + bp_ref[...]
  a_ref[...] = tabs[:, :DP]
  s, d = _pack_tables(tabs)
  s_ref[...] = s
  d_ref[...] = jnp.where(pl.program_id(0) < NSTEPS, d, 0.0)


def _hfin_call(hprev, hnew, st, g, b, wp, bp):
  return pl.pallas_call(
      _hfin_body,
      grid=(NSTEPS + 1,),
      in_specs=[
          pl.BlockSpec((BN_NODE, DP),
                       lambda i: (jnp.minimum(i, NSTEPS - 1), 0)),
          pl.BlockSpec((BN_NODE, DP),
                       lambda i: (jnp.minimum(i, NSTEPS - 1), 0)),
          pl.BlockSpec((2, DP), lambda i: (0, 0)),
          pl.BlockSpec((1, DP), lambda i: (0, 0)),
          pl.BlockSpec((1, DP), lambda i: (0, 0)),
          pl.BlockSpec((DP, 4 * DP), lambda i: (0, 0)),
          pl.BlockSpec((1, 4 * DP), lambda i: (0, 0)),
      ],
      out_specs=[
          pl.BlockSpec((BN_NODE, DP),
                       lambda i: (jnp.minimum(i, NSTEPS - 1), 0)),
          pl.BlockSpec((BN_NODE, DP),
                       lambda i: (jnp.minimum(i, NSTEPS - 1), 0)),
          pl.BlockSpec((G, BN_NODE, 32),
                       lambda i: (0, jnp.minimum(i, NSTEPS - 1), 0)),
          pl.BlockSpec((G, BN_NODE, 16), lambda i: (0, i, 0)),
      ],
      out_shape=[
          jax.ShapeDtypeStruct((N, DP), jnp.float32),
          jax.ShapeDtypeStruct((N, DP), jnp.float32),
          jax.ShapeDtypeStruct((G, N, 32), jnp.float32),
          jax.ShapeDtypeStruct((G, NTT, 16), jnp.float32),
      ],
  )(hprev, hnew, st, g, b, wp, bp)


BE = 2048
ESTEPS = EPAD // BE

ER = E // BK
EPR = EPAD // BK


def _edgepad_body(src_ref, dst_ref, ef_ref, sn_ref,
                  srcp_ref, dstp_ref, efp_ref, snp_ref):
  srcp_ref[0:ER, :] = src_ref[...]
  srcp_ref[ER:EPR, :] = jnp.zeros((EPR - ER, BK), jnp.int32)
  dstp_ref[0:ER, :] = dst_ref[...]
  dstp_ref[ER:EPR, :] = jnp.full((EPR - ER, BK), N, jnp.int32)
  efp_ref[0:ER, :] = ef_ref[...]
  efp_ref[ER:EPR, :] = jnp.zeros((EPR - ER, BK), jnp.float32)
  sn_ref2 = sn_ref[...]
  snp_ref[0:ER, :] = sn_ref2
  snp_ref[ER:EPR, :] = jnp.zeros((EPR - ER, BK), jnp.float32)


def _edgepad_call(src, dst, ef, sn):
  outs = pl.pallas_call(
      _edgepad_body,
      out_shape=[
          jax.ShapeDtypeStruct((EPR, BK), jnp.int32),
          jax.ShapeDtypeStruct((EPR, BK), jnp.int32),
          jax.ShapeDtypeStruct((EPR, BK), jnp.float32),
          jax.ShapeDtypeStruct((EPR, BK), jnp.float32),
      ],
  )(src.reshape(ER, BK), dst.reshape(ER, BK),
    ef.reshape(ER, BK), sn.reshape(ER, BK))
  return [o.reshape(NW, NBLK, BK) for o in outs]


def _ce2_body(t_ref, ef_ref, ste_ref, werow_ref, be_ref, g1_ref, b1_ref,
              cw_ref, cb_ref, out_ref):
  m = ste_ref[0:1, :] / E
  v = ste_ref[1:2, :] / E - m * m
  bn = g1_ref[...] * (t_ref[...] - m) * lax.rsqrt(v + 1e-5) + b1_ref[...]
  e1 = ef_ref[...] * werow_ref[...] + be_ref[...] + jnp.maximum(bn, 0.0)
  out_ref[...] = jnp.dot(e1, cw_ref[...],
                         preferred_element_type=jnp.float32) + cb_ref[...]


def _ce2_call(t, ef, ste, werow, be, g1, b1, cw, cb):
  return pl.pallas_call(
      _ce2_body,
      grid=(ESTEPS,),
      in_specs=[
          pl.BlockSpec((BE, DP), lambda i: (i, 0)),
          pl.BlockSpec((BE, 1), lambda i: (i, 0)),
          pl.BlockSpec((2, DP), lambda i: (0, 0)),
          pl.BlockSpec((1, DP), lambda i: (0, 0)),
          pl.BlockSpec((1, DP), lambda i: (0, 0)),
          pl.BlockSpec((1, DP), lambda i: (0, 0)),
          pl.BlockSpec((1, DP), lambda i: (0, 0)),
          pl.BlockSpec((DP, DP), lambda i: (0, 0)),
          pl.BlockSpec((1, DP), lambda i: (0, 0)),
      ],
      out_specs=pl.BlockSpec((BE, DP), lambda i: (i, 0)),
      out_shape=jax.ShapeDtypeStruct((EPAD, DP), jnp.float32),
  )(t, ef, ste, werow, be, g1, b1, cw, cb)


def _final_body(hprev_ref, hnew_ref, st_ref, g_ref, b_ref, out_ref, acc_ref):
  i = pl.program_id(0)
  m = st_ref[0:1, :] / N
  v = st_ref[1:2, :] / N - m * m
  hn = g_ref[...] * (hnew_ref[...] - m) * lax.rsqrt(v + 1e-5) + b_ref[...]
  h2 = hprev_ref[...] + jnp.maximum(hn, 0.0)

  @pl.when(i == 0)
  def _():
    acc_ref[...] = jnp.zeros_like(acc_ref)

  acc_ref[...] += jnp.sum(h2, axis=0, keepdims=True)

  @pl.when(i == NSTEPS - 1)
  def _():
    out_ref[...] = acc_ref[...] / N


def _final_call(hprev, hnew, st, g, b):
  return pl.pallas_call(
      _final_body,
      grid=(NSTEPS,),
      in_specs=[
          pl.BlockSpec((BN_NODE, DP), lambda i: (i, 0)),
          pl.BlockSpec((BN_NODE, DP), lambda i: (i, 0)),
          pl.BlockSpec((2, DP), lambda i: (0, 0)),
          pl.BlockSpec((1, DP), lambda i: (0, 0)),
          pl.BlockSpec((1, DP), lambda i: (0, 0)),
      ],
      out_specs=pl.BlockSpec((1, DP), lambda i: (0, 0)),
      out_shape=jax.ShapeDtypeStruct((1, DP), jnp.float32),
      scratch_shapes=[pltpu.VMEM((1, DP), jnp.float32)],
  )(hprev, hnew, st, g, b)


@functools.cache
def _sc_mesh():
  return plsc.VectorSubcoreMesh(core_axis_name="c", subcore_axis_name="s",
                                num_cores=2, num_subcores=16)


def _sc_layer1(srcp, dstp, efp, snp, stab, dtab, u, zrow):
  out_type = (
      jax.ShapeDtypeStruct((2, G, NT, 32), jnp.float32),
      jax.ShapeDtypeStruct((EPAD, DP), jnp.float32),
      jax.ShapeDtypeStruct((2, 16, G, 32), jnp.float32),
  )
  scratch = [
      pltpu.VMEM_SHARED((NT, 32), jnp.float32),
      [pltpu.VMEM((BK,), jnp.int32) for _ in range(2)],
      [pltpu.VMEM((BK,), jnp.int32) for _ in range(2)],
      [pltpu.VMEM((BK,), jnp.float32) for _ in range(2)],
      [pltpu.VMEM((BK,), jnp.float32) for _ in range(2)],
      [pltpu.VMEM((BK,), jnp.int32) for _ in range(2)],
      [pltpu.VMEM((BK,), jnp.int32) for _ in range(2)],
      [pltpu.VMEM((BK,), jnp.float32) for _ in range(2)],
      [pltpu.VMEM((BK,), jnp.float32) for _ in range(2)],
      [pltpu.VMEM((BK, 32), jnp.float32) for _ in range(2)],
      [pltpu.VMEM((BK, 16), jnp.float32) for _ in range(2)],
      [pltpu.VMEM((BK, 32), jnp.float32) for _ in range(2)],
      [pltpu.VMEM((BK, 16), jnp.float32) for _ in range(2)],
      pltpu.VMEM((DP,), jnp.float32),
      pltpu.VMEM((32,), jnp.float32),
      [pltpu.SemaphoreType.DMA for _ in range(2)],
      [pltpu.SemaphoreType.DMA for _ in range(2)],
      [pltpu.SemaphoreType.DMA for _ in range(2)],
      [pltpu.SemaphoreType.DMA for _ in range(2)],
  ]

  @functools.partial(pl.kernel, out_type=out_type, mesh=_sc_mesh(),
                     scratch_types=scratch,
                     compiler_params=pltpu.CompilerParams(
                         use_tc_tiling_on_sc=False,
                         needs_layout_passes=False))
  def body(srcp_h, dstp_h, efp_h, snp_h, stab_h, dtab_h, u_h, z_h,
           nd_h, t_h, st_h,
           acc, SIDXL, DIDXL, EFL, SNL, SGI, DSTS, EFS, SNS,
           SV, DV, PV, TV, uv, stbuf, sl, sg, ss, st):
    c = lax.axis_index("c")
    s = lax.axis_index("s")
    wid = c * 16 + s
    ebase = wid * EPT
    pltpu.sync_copy(u_h, uv)
    z16 = jnp.zeros((16,), jnp.float32)

    for g in range(G):
      pltpu.sync_copy(z_h, acc.at[pl.ds(s * RPS, RPS)])
      plsc.subcore_barrier()
      ug = uv[pl.ds(16 * g, 16)]

      def t_dst(bb):
        return t_h.at[pl.ds(ebase + bb * BK, BK), pl.ds(16 * g, 16)]

      def body2(j, carry):
        for par in range(2):
          bb = 2 * j + par
          p_i = (par + 1) % 2
          p_c = par

          @pl.when(bb < NBLK)
          def _():
            pltpu.async_copy(srcp_h.at[wid, bb], SIDXL[p_c], sl[p_c])
            pltpu.async_copy(dstp_h.at[wid, bb], DIDXL[p_c], sl[p_c])
            pltpu.async_copy(efp_h.at[wid, bb], EFL[p_c], sl[p_c])
            pltpu.async_copy(snp_h.at[wid, bb], SNL[p_c], sl[p_c])

          @pl.when(jnp.logical_and(bb >= 3, bb < NBLK + 3))
          def _():
            pltpu.make_async_copy(PV[p_i], acc.at[DSTS[p_i]], ss[p_i]).wait()
            pltpu.make_async_copy(TV[p_i], t_dst(bb - 3), st[p_i]).wait()

          @pl.when(jnp.logical_and(bb >= 1, bb < NBLK + 1))
          def _():
            pltpu.make_async_copy(
                srcp_h.at[wid, bb - 1], SIDXL[p_i], sl[p_i]).wait()
            pltpu.make_async_copy(
                dstp_h.at[wid, bb - 1], DIDXL[p_i], sl[p_i]).wait()
            pltpu.make_async_copy(
                efp_h.at[wid, bb - 1], EFL[p_i], sl[p_i]).wait()
            pltpu.make_async_copy(
                snp_h.at[wid, bb - 1], SNL[p_i], sl[p_i]).wait()
            for kk in range(BK // 16):
              sl16 = pl.ds(kk * 16, 16)
              SGI[p_i][sl16] = SIDXL[p_i][sl16]
              DSTS[p_i][sl16] = DIDXL[p_i][sl16]
              EFS[p_i][sl16] = EFL[p_i][sl16]
              SNS[p_i][sl16] = SNL[p_i][sl16]
            pltpu.async_copy(stab_h.at[g].at[SGI[p_i]], SV[p_i], sg[p_i])
            pltpu.async_copy(dtab_h.at[g].at[DSTS[p_i]], DV[p_i], sg[p_i])

          def c_stage(ec):
            pltpu.make_async_copy(
                stab_h.at[g].at[SGI[p_c]], SV[p_c], sg[p_c]).wait()
            pltpu.make_async_copy(
                dtab_h.at[g].at[DSTS[p_c]], DV[p_c], sg[p_c]).wait()

            @plsc.parallel_loop(0, BK, unroll=4, carry=ec)
            def edge_loop(i, ec2):
              ssum, ssq = ec2
              bh = SV[p_c][i, pl.ds(0, 16)]
              dh = SV[p_c][i, pl.ds(16, 16)]
              eh = DV[p_c][i, pl.ds(0, 16)]
              ifull = jnp.full((16,), i, jnp.int32)
              efb = plsc.load_gather(EFS[p_c], [ifull])
              snb = plsc.load_gather(SNS[p_c], [ifull])
              en = efb * ug + dh + eh
              sig = 1.0 / (1.0 + jnp.exp(-en))
              PV[p_c][i, pl.ds(0, 16)] = sig * bh
              PV[p_c][i, pl.ds(16, 16)] = sig
              t = en * snb
              TV[p_c][i, pl.ds(0, 16)] = t
              return ssum + t, ssq + t * t

            ec = edge_loop
            pltpu.async_copy(PV[p_c], acc.at[DSTS[p_c]], ss[p_c], add=True)
            pltpu.async_copy(TV[p_c], t_dst(bb - 2), st[p_c])
            return ec

          carry = lax.cond(
              jnp.logical_and(bb >= 2, bb < NBLK + 2),
              c_stage, lambda ec: ec, carry)
        return carry

      ssum, ssq = lax.fori_loop(0, (NBLK + 4) // 2, body2, (z16, z16))
      stbuf[pl.ds(0, 16)] = ssum
      stbuf[pl.ds(16, 16)] = ssq
      pltpu.sync_copy(stbuf, st_h.at[c, s, g])
      plsc.subcore_barrier()
      pltpu.sync_copy(acc.at[pl.ds(s * RPS, RPS)],
                      nd_h.at[c, g, pl.ds(s * RPS, RPS)])

  return body(srcp, dstp, efp, snp, stab, dtab, u, zrow)


def _sc_layer2(srcp, dstp, ce, stab, dtab, zrow):
  out_type = jax.ShapeDtypeStruct((2, G, NT, 32), jnp.float32)
  scratch = [
      pltpu.VMEM_SHARED((NT, 32), jnp.float32),
      [pltpu.VMEM((BK,), jnp.int32) for _ in range(2)],
      [pltpu.VMEM((BK,), jnp.int32) for _ in range(2)],
      [pltpu.VMEM((BK,), jnp.int32) for _ in range(2)],
      [pltpu.VMEM((BK,), jnp.int32) for _ in range(2)],
      [pltpu.VMEM((BK, 32), jnp.float32) for _ in range(2)],
      [pltpu.VMEM((BK, 16), jnp.float32) for _ in range(2)],
      [pltpu.VMEM((BK, 16), jnp.float32) for _ in range(2)],
      [pltpu.VMEM((BK, 32), jnp.float32) for _ in range(2)],
      [pltpu.SemaphoreType.DMA for _ in range(2)],
      [pltpu.SemaphoreType.DMA for _ in range(2)],
      [pltpu.SemaphoreType.DMA for _ in range(2)],
  ]

  @functools.partial(pl.kernel, out_type=out_type, mesh=_sc_mesh(),
                     scratch_types=scratch,
                     compiler_params=pltpu.CompilerParams(
                         use_tc_tiling_on_sc=False,
                         needs_layout_passes=False))
  def body(srcp_h, dstp_h, ce_h, stab_h, dtab_h, z_h, nd_h,
           acc, SIDXL, DIDXL, SGI, DSTS, SV, DV, CV, PV, sl, sg, ss):
    c = lax.axis_index("c")
    s = lax.axis_index("s")
    wid = c * 16 + s
    ebase = wid * EPT

    for g in range(G):
      pltpu.sync_copy(z_h, acc.at[pl.ds(s * RPS, RPS)])
      plsc.subcore_barrier()

      def ce_src(bb):
        return ce_h.at[pl.ds(ebase + bb * BK, BK), pl.ds(16 * g, 16)]

      def body2(j, carry):
        for par in range(2):
          bb = 2 * j + par
          p_i = (par + 1) % 2
          p_c = par

          @pl.when(bb < NBLK)
          def _():
            pltpu.async_copy(srcp_h.at[wid, bb], SIDXL[p_c], sl[p_c])
            pltpu.async_copy(dstp_h.at[wid, bb], DIDXL[p_c], sl[p_c])

          @pl.when(jnp.logical_and(bb >= 3, bb < NBLK + 3))
          def _():
            pltpu.make_async_copy(PV[p_i], acc.at[DSTS[p_i]], ss[p_i]).wait()

          @pl.when(jnp.logical_and(bb >= 1, bb < NBLK + 1))
          def _():
            pltpu.make_async_copy(
                srcp_h.at[wid, bb - 1], SIDXL[p_i], sl[p_i]).wait()
            pltpu.make_async_copy(
                dstp_h.at[wid, bb - 1], DIDXL[p_i], sl[p_i]).wait()
            for kk in range(BK // 16):
              sl16 = pl.ds(kk * 16, 16)
              SGI[p_i][sl16] = SIDXL[p_i][sl16]
              DSTS[p_i][sl16] = DIDXL[p_i][sl16]
            pltpu.async_copy(stab_h.at[g].at[SGI[p_i]], SV[p_i], sg[p_i])
            pltpu.async_copy(dtab_h.at[g].at[DSTS[p_i]], DV[p_i], sg[p_i])
            pltpu.async_copy(ce_src(bb - 1), CV[p_i], sg[p_i])

          @pl.when(jnp.logical_and(bb >= 2, bb < NBLK + 2))
          def _():
            pltpu.make_async_copy(
                stab_h.at[g].at[SGI[p_c]], SV[p_c], sg[p_c]).wait()
            pltpu.make_async_copy(
                dtab_h.at[g].at[DSTS[p_c]], DV[p_c], sg[p_c]).wait()
            pltpu.make_async_copy(ce_src(bb - 2), CV[p_c], sg[p_c]).wait()

            @plsc.parallel_loop(0, BK, unroll=4)
            def edge_loop(i):
              bh = SV[p_c][i, pl.ds(0, 16)]
              dh = SV[p_c][i, pl.ds(16, 16)]
              eh = DV[p_c][i, pl.ds(0, 16)]
              en = CV[p_c][i, pl.ds(0, 16)] + dh + eh
              sig = 1.0 / (1.0 + jnp.exp(-en))
              PV[p_c][i, pl.ds(0, 16)] = sig * bh
              PV[p_c][i, pl.ds(16, 16)] = sig

            pltpu.async_copy(PV[p_c], acc.at[DSTS[p_c]], ss[p_c], add=True)
        return carry

      lax.fori_loop(0, (NBLK + 4) // 2, body2, 0)
      plsc.subcore_barrier()
      pltpu.sync_copy(acc.at[pl.ds(s * RPS, RPS)],
                      nd_h.at[c, g, pl.ds(s * RPS, RPS)])

  return body(srcp, dstp, ce, stab, dtab, zrow)


def _padcol(a, width):
  return jnp.pad(a, ((0, 0), (0, width - a.shape[1])))


def _padvec(v, width):
  return jnp.pad(v, (0, width - v.shape[0]))


def _run(edge_index, nodes_feat, edges_feat, snorm_n, snorm_e, params):
  f32 = jnp.float32
  src = edge_index[0]
  dst = edge_index[1]

  whp = _padcol(params["Wh"], DP)
  bhp = _padvec(params["bh"], DP)[None, :]
  lp = params["layers"]

  def pack_layer(p):
    wp = jnp.concatenate(
        [_padcol(p[k + "_W"], DP) for k in "ABDE"], axis=1)
    wp = jnp.pad(wp, ((0, DP - D), (0, 0)))
    bp = jnp.concatenate([_padvec(p[k + "_b"], DP) for k in "ABDE"])[None, :]
    return wp, bp

  wp1, bp1 = pack_layer(lp[0])
  wp2, bp2 = pack_layer(lp[1])
  w1fold = _padvec(params["be"] @ lp[0]["C_W"] + lp[0]["C_b"], DP)
  bp1 = bp1.at[0, 3 * DP:4 * DP].add(w1fold)

  we_row = _padvec(params["We"][0], DP)[None, :]
  bep = _padvec(params["be"], DP)[None, :]
  u1 = _padvec(params["We"][0] @ lp[0]["C_W"], DP)
  c2w = jnp.pad(_padcol(lp[1]["C_W"], DP), ((0, DP - D), (0, 0)))
  c2b = _padvec(lp[1]["C_b"], DP)[None, :]
  bnh1_g = _padvec(lp[0]["bnh_g"], DP)[None, :]
  bnh1_b = _padvec(lp[0]["bnh_b"], DP)[None, :]
  bne1_g = _padvec(lp[0]["bne_g"], DP)[None, :]
  bne1_b = _padvec(lp[0]["bne_b"], DP)[None, :]
  bnh2_g = _padvec(lp[1]["bnh_g"], DP)[None, :]
  bnh2_b = _padvec(lp[1]["bnh_b"], DP)[None, :]

  srcp, dstp, efp, snp = _edgepad_call(src, dst, edges_feat[:, 0],
                                       snorm_e[:, 0])
  ef2d = efp.reshape(EPAD, 1)
  zrow = jnp.zeros((RPS, 32), f32)

  h0, a1, s1, d1 = _entry_call(nodes_feat, whp, bhp, wp1, bp1)
  nd1, t1, st1 = _sc_layer1(srcp, dstp, efp, snp, s1, d1, u1, zrow)
  hnew1, hst1, ste1 = _hnew_call(a1, nd1, snorm_n, st1)
  h1, a2, s2, d2 = _hfin_call(h0, hnew1, hst1, bnh1_g, bnh1_b, wp2, bp2)

  ce2 = _ce2_call(t1, ef2d, ste1, we_row, bep, bne1_g, bne1_b, c2w, c2b)
  nd2 = _sc_layer2(srcp, dstp, ce2, s2, d2, zrow)
  hnew2, hst2, _ = _hnew_call(a2, nd2, snorm_n, st1)
  out = _final_call(h1, hnew2, hst2, bnh2_g, bnh2_b)
  return out[:, :D]


_run_jit = jax.jit(_run)


def kernel(edge_index, nodes_feat, edges_feat, nodes_num_norm_sqrt,
           edges_num_norm_sqrt, params):
  return _run_jit(edge_index, nodes_feat, edges_feat, nodes_num_norm_sqrt,
              edges_num_norm_sqrt, params)


# speedup vs baseline: 4.8029x; 1.1905x over previous
import functools

import jax
import jax.numpy as jnp
from jax import lax
from jax.experimental import pallas as pl
from jax.experimental.pallas import tpu as pltpu
from jax.experimental.pallas import tpu_sc as plsc

N = 50000
E = 800000
IN_DIM = 64
D = 70
DP = 80
G = 5
NW = 32
BK = 128
NBLK = 196
EPT = NBLK * BK
EPAD = NW * EPT
NT = 50016
NTT = 52000
RPS = NT // 16


BN_NODE = 2000
NSTEPS = N // BN_NODE


def _pack_tables(tabs):
  Bh = tabs[:, DP:2 * DP]
  Dh = tabs[:, 2 * DP:3 * DP]
  Eh = tabs[:, 3 * DP:4 * DP]
  src = jnp.stack([
      jnp.concatenate([Bh[:, 16 * g:16 * (g + 1)], Dh[:, 16 * g:16 * (g + 1)]],
                      axis=1) for g in range(G)], axis=0)
  dst = jnp.stack([Eh[:, 16 * g:16 * (g + 1)] for g in range(G)], axis=0)
  return src, dst


def _entry_body(x_ref, wh_ref, bh_ref, wp_ref, bp_ref,
                h0_ref, a_ref, s_ref, d_ref):
  h0 = jnp.dot(x_ref[...], wh_ref[...],
               preferred_element_type=jnp.float32) + bh_ref[...]
  tabs = jnp.dot(h0, wp_ref[...],
                 preferred_element_type=jnp.float32) + bp_ref[...]
  h0_ref[...] = h0
  a_ref[...] = tabs[:, :DP]
  s, d = _pack_tables(tabs)
  s_ref[...] = s
  d_ref[...] = jnp.where(pl.program_id(0) < NSTEPS, d, 0.0)


def _entry_call(x, wh, bh, wp, bp):
  return pl.pallas_call(
      _entry_body,
      grid=(NSTEPS + 1,),
      in_specs=[
          pl.BlockSpec((BN_NODE, IN_DIM),
                       lambda i: (jnp.minimum(i, NSTEPS - 1), 0)),
          pl.BlockSpec((IN_DIM, DP), lambda i: (0, 0)),
          pl.BlockSpec((1, DP), lambda i: (0, 0)),
          pl.BlockSpec((DP, 4 * DP), lambda i: (0, 0)),
          pl.BlockSpec((1, 4 * DP), lambda i: (0, 0)),
      ],
      out_specs=[
          pl.BlockSpec((BN_NODE, DP),
                       lambda i: (jnp.minimum(i, NSTEPS - 1), 0)),
          pl.BlockSpec((BN_NODE, DP),
                       lambda i: (jnp.minimum(i, NSTEPS - 1), 0)),
          pl.BlockSpec((G, BN_NODE, 32),
                       lambda i: (0, jnp.minimum(i, NSTEPS - 1), 0)),
          pl.BlockSpec((G, BN_NODE, 16), lambda i: (0, i, 0)),
      ],
      out_shape=[
          jax.ShapeDtypeStruct((N, DP), jnp.float32),
          jax.ShapeDtypeStruct((N, DP), jnp.float32),
          jax.ShapeDtypeStruct((G, N, 32), jnp.float32),
          jax.ShapeDtypeStruct((G, NTT, 16), jnp.float32),
      ],
  )(x, wh, bh, wp, bp)


def _hnew_body(a_ref, nd_ref, snn_ref, est_ref, h_ref, st_ref, ste_ref,
               acc_ref):
  i = pl.program_id(0)
  num = jnp.concatenate(
      [nd_ref[0, g, :, 0:16] + nd_ref[1, g, :, 0:16] for g in range(G)],
      axis=1)
  den = jnp.concatenate(
      [nd_ref[0, g, :, 16:32] + nd_ref[1, g, :, 16:32] for g in range(G)],
      axis=1)
  hnew = (a_ref[...] + num / (den + 1e-6)) * snn_ref[...]
  h_ref[...] = hnew

  @pl.when(i == 0)
  def _():
    acc_ref[...] = jnp.zeros_like(acc_ref)

  acc_ref[0:1, :] += jnp.sum(hnew, axis=0, keepdims=True)
  acc_ref[1:2, :] += jnp.sum(hnew * hnew, axis=0, keepdims=True)

  @pl.when(i == NSTEPS - 1)
  def _():
    st_ref[...] = acc_ref[...]
    sums = jnp.concatenate(
        [jnp.sum(est_ref[:, :, g, 0:16], axis=(0, 1)) for g in range(G)])
    sqs = jnp.concatenate(
        [jnp.sum(est_ref[:, :, g, 16:32], axis=(0, 1)) for g in range(G)])
    ste_ref[...] = jnp.stack([sums, sqs], axis=0)


def _hnew_call(a, nd, snn, est):
  return pl.pallas_call(
      _hnew_body,
      grid=(NSTEPS,),
      in_specs=[
          pl.BlockSpec((BN_NODE, DP), lambda i: (i, 0)),
          pl.BlockSpec((2, G, BN_NODE, 32), lambda i: (0, 0, i, 0)),
          pl.BlockSpec((BN_NODE, 1), lambda i: (i, 0)),
          pl.BlockSpec((2, 16, G, 32), lambda i: (0, 0, 0, 0)),
      ],
      out_specs=[
          pl.BlockSpec((BN_NODE, DP), lambda i: (i, 0)),
          pl.BlockSpec((2, DP), lambda i: (0, 0)),
          pl.BlockSpec((2, DP), lambda i: (0, 0)),
      ],
      out_shape=[
          jax.ShapeDtypeStruct((N, DP), jnp.float32),
          jax.ShapeDtypeStruct((2, DP), jnp.float32),
          jax.ShapeDtypeStruct((2, DP), jnp.float32),
      ],
      scratch_shapes=[pltpu.VMEM((2, DP), jnp.float32)],
  )(a, nd, snn, est)


def _hfin_body(hprev_ref, hnew_ref, st_ref, g_ref, b_ref, wp_ref, bp_ref,
               h_ref, a_ref, s_ref, d_ref):
  m = st_ref[0:1, :] / N
  v = st_ref[1:2, :] / N - m * m
  hn = g_ref[...] * (hnew_ref[...] - m) * lax.rsqrt(v + 1e-5) + b_ref[...]
  h1 = hprev_ref[...] + jnp.maximum(hn, 0.0)
  h_ref[...] = h1
  tabs = jnp.dot(h1, wp_ref[...],
                 preferred_element_type=jnp.float32) + bp_ref[...]
  a_ref[...] = tabs[:, :DP]
  s, d = _pack_tables(tabs)
  s_ref[...] = s
  d_ref[...] = jnp.where(pl.program_id(0) < NSTEPS, d, 0.0)


def _hfin_call(hprev, hnew, st, g, b, wp, bp):
  return pl.pallas_call(
      _hfin_body,
      grid=(NSTEPS + 1,),
      in_specs=[
          pl.BlockSpec((BN_NODE, DP),
                       lambda i: (jnp.minimum(i, NSTEPS - 1), 0)),
          pl.BlockSpec((BN_NODE, DP),
                       lambda i: (jnp.minimum(i, NSTEPS - 1), 0)),
          pl.BlockSpec((2, DP), lambda i: (0, 0)),
          pl.BlockSpec((1, DP), lambda i: (0, 0)),
          pl.BlockSpec((1, DP), lambda i: (0, 0)),
          pl.BlockSpec((DP, 4 * DP), lambda i: (0, 0)),
          pl.BlockSpec((1, 4 * DP), lambda i: (0, 0)),
      ],
      out_specs=[
          pl.BlockSpec((BN_NODE, DP),
                       lambda i: (jnp.minimum(i, NSTEPS - 1), 0)),
          pl.BlockSpec((BN_NODE, DP),
                       lambda i: (jnp.minimum(i, NSTEPS - 1), 0)),
          pl.BlockSpec((G, BN_NODE, 32),
                       lambda i: (0, jnp.minimum(i, NSTEPS - 1), 0)),
          pl.BlockSpec((G, BN_NODE, 16), lambda i: (0, i, 0)),
      ],
      out_shape=[
          jax.ShapeDtypeStruct((N, DP), jnp.float32),
          jax.ShapeDtypeStruct((N, DP), jnp.float32),
          jax.ShapeDtypeStruct((G, N, 32), jnp.float32),
          jax.ShapeDtypeStruct((G, NTT, 16), jnp.float32),
      ],
  )(hprev, hnew, st, g, b, wp, bp)


BE = 2048
ESTEPS = EPAD // BE

ER = E // BK
EPR = EPAD // BK


def _edgepad_body(src_ref, dst_ref, ef_ref, sn_ref,
                  srcp_ref, dstp_ref, efp_ref, snp_ref):
  srcp_ref[0:ER, :] = src_ref[...]
  srcp_ref[ER:EPR, :] = jnp.zeros((EPR - ER, BK), jnp.int32)
  dstp_ref[0:ER, :] = dst_ref[...]
  dstp_ref[ER:EPR, :] = jnp.full((EPR - ER, BK), N, jnp.int32)
  efp_ref[0:ER, :] = ef_ref[...]
  efp_ref[ER:EPR, :] = jnp.zeros((EPR - ER, BK), jnp.float32)
  sn_ref2 = sn_ref[...]
  snp_ref[0:ER, :] = sn_ref2
  snp_ref[ER:EPR, :] = jnp.zeros((EPR - ER, BK), jnp.float32)


def _edgepad_call(src, dst, ef, sn):
  outs = pl.pallas_call(
      _edgepad_body,
      out_shape=[
          jax.ShapeDtypeStruct((EPR, BK), jnp.int32),
          jax.ShapeDtypeStruct((EPR, BK), jnp.int32),
          jax.ShapeDtypeStruct((EPR, BK), jnp.float32),
          jax.ShapeDtypeStruct((EPR, BK), jnp.float32),
      ],
  )(src.reshape(ER, BK), dst.reshape(ER, BK),
    ef.reshape(ER, BK), sn.reshape(ER, BK))
  return [o.reshape(NW, NBLK, BK) for o in outs]


def _ce2_body(t_ref, ef_ref, ste_ref, werow_ref, be_ref, g1_ref, b1_ref,
              cw_ref, cb_ref, out_ref):
  m = ste_ref[0:1, :] / E
  v = ste_ref[1:2, :] / E - m * m
  t = t_ref[:, :DP]
  bn = g1_ref[...] * (t - m) * lax.rsqrt(v + 1e-5) + b1_ref[...]
  e1 = ef_ref[...] * werow_ref[...] + be_ref[...] + jnp.maximum(bn, 0.0)
  out_ref[...] = jnp.dot(e1, cw_ref[...],
                         preferred_element_type=jnp.float32) + cb_ref[...]


def _ce2_call(t, ef, ste, werow, be, g1, b1, cw, cb):
  return pl.pallas_call(
      _ce2_body,
      grid=(ESTEPS,),
      in_specs=[
          pl.BlockSpec((BE, 128), lambda i: (i, 0)),
          pl.BlockSpec((BE, 1), lambda i: (i, 0)),
          pl.BlockSpec((2, DP), lambda i: (0, 0)),
          pl.BlockSpec((1, DP), lambda i: (0, 0)),
          pl.BlockSpec((1, DP), lambda i: (0, 0)),
          pl.BlockSpec((1, DP), lambda i: (0, 0)),
          pl.BlockSpec((1, DP), lambda i: (0, 0)),
          pl.BlockSpec((DP, 128), lambda i: (0, 0)),
          pl.BlockSpec((1, 128), lambda i: (0, 0)),
      ],
      out_specs=pl.BlockSpec((BE, 128), lambda i: (i, 0)),
      out_shape=jax.ShapeDtypeStruct((EPAD, 128), jnp.float32),
  )(t, ef, ste, werow, be, g1, b1, cw, cb)


def _final_body(hprev_ref, hnew_ref, st_ref, g_ref, b_ref, out_ref, acc_ref):
  i = pl.program_id(0)
  m = st_ref[0:1, :] / N
  v = st_ref[1:2, :] / N - m * m
  hn = g_ref[...] * (hnew_ref[...] - m) * lax.rsqrt(v + 1e-5) + b_ref[...]
  h2 = hprev_ref[...] + jnp.maximum(hn, 0.0)

  @pl.when(i == 0)
  def _():
    acc_ref[...] = jnp.zeros_like(acc_ref)

  acc_ref[...] += jnp.sum(h2, axis=0, keepdims=True)

  @pl.when(i == NSTEPS - 1)
  def _():
    out_ref[...] = acc_ref[...] / N


def _final_call(hprev, hnew, st, g, b):
  return pl.pallas_call(
      _final_body,
      grid=(NSTEPS,),
      in_specs=[
          pl.BlockSpec((BN_NODE, DP), lambda i: (i, 0)),
          pl.BlockSpec((BN_NODE, DP), lambda i: (i, 0)),
          pl.BlockSpec((2, DP), lambda i: (0, 0)),
          pl.BlockSpec((1, DP), lambda i: (0, 0)),
          pl.BlockSpec((1, DP), lambda i: (0, 0)),
      ],
      out_specs=pl.BlockSpec((1, DP), lambda i: (0, 0)),
      out_shape=jax.ShapeDtypeStruct((1, DP), jnp.float32),
      scratch_shapes=[pltpu.VMEM((1, DP), jnp.float32)],
  )(hprev, hnew, st, g, b)


@functools.cache
def _sc_mesh():
  return plsc.VectorSubcoreMesh(core_axis_name="c", subcore_axis_name="s",
                                num_cores=2, num_subcores=16)


def _sc_layer1(srcp, dstp, efp, snp, stab, dtab, u, zrow):
  out_type = (
      jax.ShapeDtypeStruct((2, G, NT, 32), jnp.float32),
      jax.ShapeDtypeStruct((EPAD, 128), jnp.float32),
      jax.ShapeDtypeStruct((2, 16, G, 32), jnp.float32),
  )
  scratch = [
      pltpu.VMEM_SHARED((NT, 32), jnp.float32),
      [pltpu.VMEM((BK,), jnp.int32) for _ in range(2)],
      [pltpu.VMEM((BK,), jnp.int32) for _ in range(2)],
      [pltpu.VMEM((BK,), jnp.float32) for _ in range(2)],
      [pltpu.VMEM((BK,), jnp.float32) for _ in range(2)],
      [pltpu.VMEM((BK,), jnp.int32) for _ in range(2)],
      [pltpu.VMEM((BK,), jnp.int32) for _ in range(2)],
      [pltpu.VMEM((BK,), jnp.float32) for _ in range(2)],
      [pltpu.VMEM((BK,), jnp.float32) for _ in range(2)],
      [pltpu.VMEM((BK, 32), jnp.float32) for _ in range(2)],
      [pltpu.VMEM((BK, 16), jnp.float32) for _ in range(2)],
      [pltpu.VMEM((BK, 32), jnp.float32) for _ in range(2)],
      [pltpu.VMEM((BK, 16), jnp.float32) for _ in range(2)],
      pltpu.VMEM((DP,), jnp.float32),
      pltpu.VMEM((32,), jnp.float32),
      [pltpu.SemaphoreType.DMA for _ in range(2)],
      [pltpu.SemaphoreType.DMA for _ in range(2)],
      [pltpu.SemaphoreType.DMA for _ in range(2)],
      [pltpu.SemaphoreType.DMA for _ in range(2)],
  ]

  @functools.partial(pl.kernel, out_type=out_type, mesh=_sc_mesh(),
                     scratch_types=scratch,
                     compiler_params=pltpu.CompilerParams(
                         use_tc_tiling_on_sc=False,
                         needs_layout_passes=False))
  def body(srcp_h, dstp_h, efp_h, snp_h, stab_h, dtab_h, u_h, z_h,
           nd_h, t_h, st_h,
           acc, SIDXL, DIDXL, EFL, SNL, SGI, DSTS, EFS, SNS,
           SV, DV, PV, TV, uv, stbuf, sl, sg, ss, st):
    c = lax.axis_index("c")
    s = lax.axis_index("s")
    wid = c * 16 + s
    ebase = wid * EPT
    pltpu.sync_copy(u_h, uv)
    z16 = jnp.zeros((16,), jnp.float32)

    for g in range(G):
      pltpu.sync_copy(z_h, acc.at[pl.ds(s * RPS, RPS)])
      plsc.subcore_barrier()
      ug = uv[pl.ds(16 * g, 16)]

      def t_dst(bb):
        return t_h.at[pl.ds(ebase + bb * BK, BK), pl.ds(16 * g, 16)]

      def body2(j, carry):
        for par in range(2):
          bb = 2 * j + par
          p_i = (par + 1) % 2
          p_c = par

          @pl.when(bb < NBLK)
          def _():
            pltpu.async_copy(srcp_h.at[wid, bb], SIDXL[p_c], sl[p_c])
            pltpu.async_copy(dstp_h.at[wid, bb], DIDXL[p_c], sl[p_c])
            pltpu.async_copy(efp_h.at[wid, bb], EFL[p_c], sl[p_c])
            pltpu.async_copy(snp_h.at[wid, bb], SNL[p_c], sl[p_c])

          @pl.when(jnp.logical_and(bb >= 3, bb < NBLK + 3))
          def _():
            pltpu.make_async_copy(PV[p_i], acc.at[DSTS[p_i]], ss[p_i]).wait()
            pltpu.make_async_copy(TV[p_i], t_dst(bb - 3), st[p_i]).wait()

          @pl.when(jnp.logical_and(bb >= 1, bb < NBLK + 1))
          def _():
            pltpu.make_async_copy(
                srcp_h.at[wid, bb - 1], SIDXL[p_i], sl[p_i]).wait()
            pltpu.make_async_copy(
                dstp_h.at[wid, bb - 1], DIDXL[p_i], sl[p_i]).wait()
            pltpu.make_async_copy(
                efp_h.at[wid, bb - 1], EFL[p_i], sl[p_i]).wait()
            pltpu.make_async_copy(
                snp_h.at[wid, bb - 1], SNL[p_i], sl[p_i]).wait()
            for kk in range(BK // 16):
              sl16 = pl.ds(kk * 16, 16)
              SGI[p_i][sl16] = SIDXL[p_i][sl16]
              DSTS[p_i][sl16] = DIDXL[p_i][sl16]
              EFS[p_i][sl16] = EFL[p_i][sl16]
              SNS[p_i][sl16] = SNL[p_i][sl16]
            pltpu.async_copy(stab_h.at[g].at[SGI[p_i]], SV[p_i], sg[p_i])
            pltpu.async_copy(dtab_h.at[g].at[DSTS[p_i]], DV[p_i], sg[p_i])

          def c_stage(ec):
            pltpu.make_async_copy(
                stab_h.at[g].at[SGI[p_c]], SV[p_c], sg[p_c]).wait()
            pltpu.make_async_copy(
                dtab_h.at[g].at[DSTS[p_c]], DV[p_c], sg[p_c]).wait()

            @plsc.parallel_loop(0, BK, unroll=4, carry=ec)
            def edge_loop(i, ec2):
              ssum, ssq = ec2
              bh = SV[p_c][i, pl.ds(0, 16)]
              dh = SV[p_c][i, pl.ds(16, 16)]
              eh = DV[p_c][i, pl.ds(0, 16)]
              ifull = jnp.full((16,), i, jnp.int32)
              efb = plsc.load_gather(EFS[p_c], [ifull])
              snb = plsc.load_gather(SNS[p_c], [ifull])
              en = efb * ug + dh + eh
              sig = 1.0 / (1.0 + jnp.exp(-en))
              PV[p_c][i, pl.ds(0, 16)] = sig * bh
              PV[p_c][i, pl.ds(16, 16)] = sig
              t = en * snb
              TV[p_c][i, pl.ds(0, 16)] = t
              return ssum + t, ssq + t * t

            ec = edge_loop
            pltpu.async_copy(PV[p_c], acc.at[DSTS[p_c]], ss[p_c], add=True)
            pltpu.async_copy(TV[p_c], t_dst(bb - 2), st[p_c])
            return ec

          carry = lax.cond(
              jnp.logical_and(bb >= 2, bb < NBLK + 2),
              c_stage, lambda ec: ec, carry)
        return carry

      ssum, ssq = lax.fori_loop(0, (NBLK + 4) // 2, body2, (z16, z16))
      stbuf[pl.ds(0, 16)] = ssum
      stbuf[pl.ds(16, 16)] = ssq
      pltpu.sync_copy(stbuf, st_h.at[c, s, g])
      plsc.subcore_barrier()
      pltpu.sync_copy(acc.at[pl.ds(s * RPS, RPS)],
                      nd_h.at[c, g, pl.ds(s * RPS, RPS)])

  return body(srcp, dstp, efp, snp, stab, dtab, u, zrow)


def _sc_layer2(srcp, dstp, ce, stab, dtab, zrow):
  out_type = jax.ShapeDtypeStruct((2, G, NT, 32), jnp.float32)
  scratch = [
      pltpu.VMEM_SHARED((NT, 32), jnp.float32),
      [pltpu.VMEM((BK,), jnp.int32) for _ in range(2)],
      [pltpu.VMEM((BK,), jnp.int32) for _ in range(2)],
      [pltpu.VMEM((BK,), jnp.int32) for _ in range(2)],
      [pltpu.VMEM((BK,), jnp.int32) for _ in range(2)],
      [pltpu.VMEM((BK, 32), jnp.float32) for _ in range(2)],
      [pltpu.VMEM((BK, 16), jnp.float32) for _ in range(2)],
      [pltpu.VMEM((BK, 16), jnp.float32) for _ in range(2)],
      [pltpu.VMEM((BK, 32), jnp.float32) for _ in range(2)],
      [pltpu.SemaphoreType.DMA for _ in range(2)],
      [pltpu.SemaphoreType.DMA for _ in range(2)],
      [pltpu.SemaphoreType.DMA for _ in range(2)],
  ]

  @functools.partial(pl.kernel, out_type=out_type, mesh=_sc_mesh(),
                     scratch_types=scratch,
                     compiler_params=pltpu.CompilerParams(
                         use_tc_tiling_on_sc=False,
                         needs_layout_passes=False))
  def body(srcp_h, dstp_h, ce_h, stab_h, dtab_h, z_h, nd_h,
           acc, SIDXL, DIDXL, SGI, DSTS, SV, DV, CV, PV, sl, sg, ss):
    c = lax.axis_index("c")
    s = lax.axis_index("s")
    wid = c * 16 + s
    ebase = wid * EPT

    for g in range(G):
      pltpu.sync_copy(z_h, acc.at[pl.ds(s * RPS, RPS)])
      plsc.subcore_barrier()

      def ce_src(bb):
        return ce_h.at[pl.ds(ebase + bb * BK, BK), pl.ds(16 * g, 16)]

      def body2(j, carry):
        for par in range(2):
          bb = 2 * j + par
          p_i = (par + 1) % 2
          p_c = par

          @pl.when(bb < NBLK)
          def _():
            pltpu.async_copy(srcp_h.at[wid, bb], SIDXL[p_c], sl[p_c])
            pltpu.async_copy(dstp_h.at[wid, bb], DIDXL[p_c], sl[p_c])

          @pl.when(jnp.logical_and(bb >= 3, bb < NBLK + 3))
          def _():
            pltpu.make_async_copy(PV[p_i], acc.at[DSTS[p_i]], ss[p_i]).wait()

          @pl.when(jnp.logical_and(bb >= 1, bb < NBLK + 1))
          def _():
            pltpu.make_async_copy(
                srcp_h.at[wid, bb - 1], SIDXL[p_i], sl[p_i]).wait()
            pltpu.make_async_copy(
                dstp_h.at[wid, bb - 1], DIDXL[p_i], sl[p_i]).wait()
            for kk in range(BK // 16):
              sl16 = pl.ds(kk * 16, 16)
              SGI[p_i][sl16] = SIDXL[p_i][sl16]
              DSTS[p_i][sl16] = DIDXL[p_i][sl16]
            pltpu.async_copy(stab_h.at[g].at[SGI[p_i]], SV[p_i], sg[p_i])
            pltpu.async_copy(dtab_h.at[g].at[DSTS[p_i]], DV[p_i], sg[p_i])
            pltpu.async_copy(ce_src(bb - 1), CV[p_i], sg[p_i])

          @pl.when(jnp.logical_and(bb >= 2, bb < NBLK + 2))
          def _():
            pltpu.make_async_copy(
                stab_h.at[g].at[SGI[p_c]], SV[p_c], sg[p_c]).wait()
            pltpu.make_async_copy(
                dtab_h.at[g].at[DSTS[p_c]], DV[p_c], sg[p_c]).wait()
            pltpu.make_async_copy(ce_src(bb - 2), CV[p_c], sg[p_c]).wait()

            @plsc.parallel_loop(0, BK, unroll=4)
            def edge_loop(i):
              bh = SV[p_c][i, pl.ds(0, 16)]
              dh = SV[p_c][i, pl.ds(16, 16)]
              eh = DV[p_c][i, pl.ds(0, 16)]
              en = CV[p_c][i, pl.ds(0, 16)] + dh + eh
              sig = 1.0 / (1.0 + jnp.exp(-en))
              PV[p_c][i, pl.ds(0, 16)] = sig * bh
              PV[p_c][i, pl.ds(16, 16)] = sig

            pltpu.async_copy(PV[p_c], acc.at[DSTS[p_c]], ss[p_c], add=True)
        return carry

      lax.fori_loop(0, (NBLK + 4) // 2, body2, 0)
      plsc.subcore_barrier()
      pltpu.sync_copy(acc.at[pl.ds(s * RPS, RPS)],
                      nd_h.at[c, g, pl.ds(s * RPS, RPS)])

  return body(srcp, dstp, ce, stab, dtab, zrow)


def _padcol(a, width):
  return jnp.pad(a, ((0, 0), (0, width - a.shape[1])))


def _padvec(v, width):
  return jnp.pad(v, (0, width - v.shape[0]))


def _run(edge_index, nodes_feat, edges_feat, snorm_n, snorm_e, params):
  f32 = jnp.float32
  src = edge_index[0]
  dst = edge_index[1]

  whp = _padcol(params["Wh"], DP)
  bhp = _padvec(params["bh"], DP)[None, :]
  lp = params["layers"]

  def pack_layer(p):
    wp = jnp.concatenate(
        [_padcol(p[k + "_W"], DP) for k in "ABDE"], axis=1)
    wp = jnp.pad(wp, ((0, DP - D), (0, 0)))
    bp = jnp.concatenate([_padvec(p[k + "_b"], DP) for k in "ABDE"])[None, :]
    return wp, bp

  wp1, bp1 = pack_layer(lp[0])
  wp2, bp2 = pack_layer(lp[1])
  w1fold = _padvec(params["be"] @ lp[0]["C_W"] + lp[0]["C_b"], DP)
  bp1 = bp1.at[0, 3 * DP:4 * DP].add(w1fold)

  we_row = _padvec(params["We"][0], DP)[None, :]
  bep = _padvec(params["be"], DP)[None, :]
  u1 = _padvec(params["We"][0] @ lp[0]["C_W"], DP)
  c2w = jnp.pad(_padcol(lp[1]["C_W"], 128), ((0, DP - D), (0, 0)))
  c2b = _padvec(lp[1]["C_b"], 128)[None, :]
  bnh1_g = _padvec(lp[0]["bnh_g"], DP)[None, :]
  bnh1_b = _padvec(lp[0]["bnh_b"], DP)[None, :]
  bne1_g = _padvec(lp[0]["bne_g"], DP)[None, :]
  bne1_b = _padvec(lp[0]["bne_b"], DP)[None, :]
  bnh2_g = _padvec(lp[1]["bnh_g"], DP)[None, :]
  bnh2_b = _padvec(lp[1]["bnh_b"], DP)[None, :]

  srcp, dstp, efp, snp = _edgepad_call(src, dst, edges_feat[:, 0],
                                       snorm_e[:, 0])
  ef2d = efp.reshape(EPAD, 1)
  zrow = jnp.zeros((RPS, 32), f32)

  h0, a1, s1, d1 = _entry_call(nodes_feat, whp, bhp, wp1, bp1)
  nd1, t1, st1 = _sc_layer1(srcp, dstp, efp, snp, s1, d1, u1, zrow)
  hnew1, hst1, ste1 = _hnew_call(a1, nd1, snorm_n, st1)
  h1, a2, s2, d2 = _hfin_call(h0, hnew1, hst1, bnh1_g, bnh1_b, wp2, bp2)

  ce2 = _ce2_call(t1, ef2d, ste1, we_row, bep, bne1_g, bne1_b, c2w, c2b)
  nd2 = _sc_layer2(srcp, dstp, ce2, s2, d2, zrow)
  hnew2, hst2, _ = _hnew_call(a2, nd2, snorm_n, st1)
  out = _final_call(h1, hnew2, hst2, bnh2_g, bnh2_b)
  return out[:, :D]


_run_jit = jax.jit(_run)


def kernel(edge_index, nodes_feat, edges_feat, nodes_num_norm_sqrt,
           edges_num_norm_sqrt, params):
  return _run_jit(edge_index, nodes_feat, edges_feat, nodes_num_norm_sqrt,
              edges_num_norm_sqrt, params)
